# Initial kernel scaffold; baseline (speedup 1.0000x reference)
import jax
import jax.numpy as jnp
from jax.experimental import pallas as pl


def kernel(x, nlp_features, edge_index, user_indices, W_in, b_in, W_g1, b_g1, W_g2, b_g2, W_proj, b_proj, W_p1, b_p1, W_p2, b_p2, W_p3, b_p3):
    raise NotImplementedError("write your pallas kernel here")


# speedup vs baseline: 3.0143x; 3.0143x over previous
import functools

import jax
import jax.numpy as jnp
from jax import lax
from jax.experimental import pallas as pl
from jax.experimental.pallas import tpu as pltpu
from jax.experimental.pallas import tpu_sc as plsc

N_NODES = 10000
D = 128
N_EDGES = 320000
N_USERS = 1024
NLP_DIM = 786

NC, NS = 2, 16
NW = NC * NS
EB = 80
E_PAD = NW * EB * 128
AGG_ROWS = 10240
RPT = AGG_ROWS // NS
TRASH_ROW = N_NODES
UPT = N_USERS // NS

_sc_mesh = plsc.VectorSubcoreMesh(core_axis_name="c", subcore_axis_name="s")


def _zero_accumulator(rows_v, agg_sh, s):
  def zrow(r, carry):
    def zcol(k, carry2):
      rows_v[r, pl.ds(k * 16, 16)] = jnp.zeros((16,), jnp.float32)
      return carry2
    return lax.fori_loop(0, D // 16, zcol, carry)
  lax.fori_loop(0, 128, zrow, 0)

  def zcopy(i, carry):
    pltpu.sync_copy(rows_v, agg_sh.at[pl.ds(s * RPT + i * 128, 128)])
    return carry
  lax.fori_loop(0, RPT // 128, zcopy, 0)


def _scatter_edges(h_hbm, src_hbm, dst_hbm, src_v, dst_v, rows_v, agg_sh,
                   sem, wid):
  pltpu.sync_copy(src_hbm.at[pl.ds(wid * EB, EB)], src_v)
  pltpu.sync_copy(dst_hbm.at[pl.ds(wid * EB, EB)], dst_v)

  def step(j, carry):
    pltpu.async_copy(h_hbm.at[src_v.at[j]], rows_v, sem).wait()
    pltpu.sync_copy(rows_v, agg_sh.at[dst_v.at[j]], add=True)
    return carry
  lax.fori_loop(0, EB, step, 0)


@functools.partial(
    pl.kernel,
    out_type=jax.ShapeDtypeStruct((NC, AGG_ROWS, D), jnp.float32),
    mesh=_sc_mesh,
    scratch_types=[
        pltpu.VMEM((EB, 128), jnp.int32),
        pltpu.VMEM((EB, 128), jnp.int32),
        pltpu.VMEM((128, D), jnp.float32),
        pltpu.VMEM_SHARED((AGG_ROWS, D), jnp.float32),
        pltpu.SemaphoreType.DMA,
    ],
)
def _segsum_full(h_hbm, src_hbm, dst_hbm, out_hbm,
                 src_v, dst_v, rows_v, agg_sh, sem):
  c = lax.axis_index("c")
  s = lax.axis_index("s")
  wid = c * NS + s
  _zero_accumulator(rows_v, agg_sh, s)
  plsc.subcore_barrier()
  _scatter_edges(h_hbm, src_hbm, dst_hbm, src_v, dst_v, rows_v, agg_sh,
                 sem, wid)
  plsc.subcore_barrier()
  pltpu.sync_copy(agg_sh.at[pl.ds(s * RPT, RPT)],
                  out_hbm.at[c, pl.ds(s * RPT, RPT)])


@functools.partial(
    pl.kernel,
    out_type=(jax.ShapeDtypeStruct((N_USERS, D), jnp.float32),
              jax.ShapeDtypeStruct((NC, N_USERS, D), jnp.float32)),
    mesh=_sc_mesh,
    scratch_types=[
        pltpu.VMEM((EB, 128), jnp.int32),
        pltpu.VMEM((EB, 128), jnp.int32),
        pltpu.VMEM((128, D), jnp.float32),
        pltpu.VMEM((UPT,), jnp.int32),
        pltpu.VMEM_SHARED((AGG_ROWS, D), jnp.float32),
        pltpu.SemaphoreType.DMA,
    ],
)
def _segsum_users(h_hbm, src_hbm, dst_hbm, uidx_hbm, uh_hbm, uagg_hbm,
                  src_v, dst_v, rows_v, uidx_v, agg_sh, sem):
  c = lax.axis_index("c")
  s = lax.axis_index("s")
  wid = c * NS + s
  _zero_accumulator(rows_v, agg_sh, s)
  plsc.subcore_barrier()
  _scatter_edges(h_hbm, src_hbm, dst_hbm, src_v, dst_v, rows_v, agg_sh,
                 sem, wid)
  plsc.subcore_barrier()
  pltpu.sync_copy(uidx_hbm.at[pl.ds(s * UPT, UPT)], uidx_v)
  urows_v = rows_v.at[pl.ds(0, UPT)]
  pltpu.async_copy(agg_sh.at[uidx_v], urows_v, sem).wait()
  pltpu.sync_copy(urows_v, uagg_hbm.at[c, pl.ds(s * UPT, UPT)])

  @pl.when(c == 0)
  def _():
    pltpu.async_copy(h_hbm.at[uidx_v], urows_v, sem).wait()
    pltpu.sync_copy(urows_v, uh_hbm.at[pl.ds(s * UPT, UPT)])


_NB = 10
_BR = N_NODES // _NB


def _linrelu_body(x_ref, w_ref, b_ref, o_ref):
  o_ref[...] = jnp.maximum(
      jnp.dot(x_ref[...], w_ref[...], preferred_element_type=jnp.float32)
      + b_ref[...], 0.0)


def _tc_linrelu(x, w, b):
  return pl.pallas_call(
      _linrelu_body,
      grid=(_NB,),
      in_specs=[
          pl.BlockSpec((_BR, D), lambda i: (i, 0)),
          pl.BlockSpec((D, D), lambda i: (0, 0)),
          pl.BlockSpec((1, D), lambda i: (0, 0)),
      ],
      out_specs=pl.BlockSpec((_BR, D), lambda i: (i, 0)),
      out_shape=jax.ShapeDtypeStruct((N_NODES, D), jnp.float32),
  )(x, w, b)


def _fuse_body(h_ref, a0_ref, a1_ref, w_ref, b_ref, o_ref):
  u = h_ref[...] + a0_ref[...] + a1_ref[...]
  o_ref[...] = jnp.maximum(
      jnp.dot(u, w_ref[...], preferred_element_type=jnp.float32)
      + b_ref[...], 0.0)


def _tc_fuse(h, a0, a1, w, b):
  return pl.pallas_call(
      _fuse_body,
      grid=(_NB,),
      in_specs=[
          pl.BlockSpec((_BR, D), lambda i: (i, 0)),
          pl.BlockSpec((_BR, D), lambda i: (i, 0)),
          pl.BlockSpec((_BR, D), lambda i: (i, 0)),
          pl.BlockSpec((D, D), lambda i: (0, 0)),
          pl.BlockSpec((1, D), lambda i: (0, 0)),
      ],
      out_specs=pl.BlockSpec((_BR, D), lambda i: (i, 0)),
      out_shape=jax.ShapeDtypeStruct((N_NODES, D), jnp.float32),
  )(h, a0, a1, w, b)


def _head_body(uh_ref, a0_ref, a1_ref, wg2_ref, bg2_ref, wpj_ref, bpj_ref,
               wp1a_ref, nlp_ref, wp1b_ref, bp1_ref, wp2_ref, bp2_ref,
               wp3_ref, bp3_ref, o_ref):
  f32 = jnp.float32
  u = uh_ref[...] + a0_ref[...] + a1_ref[...]
  h2 = jnp.maximum(
      jnp.dot(u, wg2_ref[...], preferred_element_type=f32) + bg2_ref[...], 0.0)
  emb = jnp.dot(h2, wpj_ref[...], preferred_element_type=f32) + bpj_ref[...]
  nz = jnp.dot(nlp_ref[...], wp1b_ref[...], preferred_element_type=f32)[0:1, :]
  z1 = jnp.maximum(
      jnp.dot(emb, wp1a_ref[...], preferred_element_type=f32)
      + nz + bp1_ref[...], 0.0)
  z2 = jnp.maximum(
      jnp.dot(z1, wp2_ref[...], preferred_element_type=f32) + bp2_ref[...],
      0.0)
  lg = jnp.dot(z2, wp3_ref[...], preferred_element_type=f32) + bp3_ref[...]
  o_ref[...] = jax.nn.sigmoid(lg)


def _tc_head(uh, a0, a1, wg2, bg2, wpj, bpj, wp1a, nlp_p, wp1b, bp1, wp2,
             bp2, wp3, bp3):
  return pl.pallas_call(
      _head_body,
      out_shape=jax.ShapeDtypeStruct((N_USERS, 128), jnp.float32),
  )(uh, a0, a1, wg2, bg2, wpj, bpj, wp1a, nlp_p, wp1b, bp1, wp2, bp2, wp3,
    bp3)


def kernel(x, nlp_features, edge_index, user_indices,
           W_in, b_in, W_g1, b_g1, W_g2, b_g2,
           W_proj, b_proj, W_p1, b_p1, W_p2, b_p2, W_p3, b_p3):
  f32 = jnp.float32
  src = edge_index[0].astype(jnp.int32)
  dst = edge_index[1].astype(jnp.int32)
  pad = E_PAD - N_EDGES
  src_p = jnp.concatenate(
      [src, jnp.zeros((pad,), jnp.int32)]).reshape(NW * EB, 128)
  dst_p = jnp.concatenate(
      [dst, jnp.full((pad,), TRASH_ROW, jnp.int32)]).reshape(NW * EB, 128)
  uidx = user_indices.astype(jnp.int32)

  h0 = _tc_linrelu(x, W_in, b_in.reshape(1, D))
  agg1 = _segsum_full(h0, src_p, dst_p)
  h1 = _tc_fuse(h0, agg1[0, :N_NODES], agg1[1, :N_NODES],
                W_g1, b_g1.reshape(1, D))
  uh1, uagg = _segsum_users(h1, src_p, dst_p, uidx)

  nlp_p = jnp.zeros((8, 896), f32).at[0, :NLP_DIM].set(nlp_features)
  wp1b = jnp.zeros((896, 256), f32).at[:NLP_DIM].set(W_p1[D:])
  wp3 = jnp.zeros((128, 128), f32).at[:, :1].set(W_p3)
  bp3 = jnp.zeros((1, 128), f32).at[0, 0].set(b_p3[0])

  out = _tc_head(uh1, uagg[0], uagg[1], W_g2, b_g2.reshape(1, D),
                 W_proj, b_proj.reshape(1, D), W_p1[:D], nlp_p, wp1b,
                 b_p1.reshape(1, 256), W_p2, b_p2.reshape(1, 128), wp3, bp3)
  return out[:, 0]


# speedup vs baseline: 3.0747x; 1.0200x over previous
import functools

import jax
import jax.numpy as jnp
from jax import lax
from jax.experimental import pallas as pl
from jax.experimental.pallas import tpu as pltpu
from jax.experimental.pallas import tpu_sc as plsc

N_NODES = 10000
D = 128
N_EDGES = 320000
N_USERS = 1024
NLP_DIM = 786

NC, NS = 2, 16
NW = NC * NS
EB = 80
E_PAD = NW * EB * 128
AGG_ROWS = 10240
RPT = AGG_ROWS // NS
TRASH_ROW = N_NODES
UPT = N_USERS // NS

_sc_mesh = plsc.VectorSubcoreMesh(core_axis_name="c", subcore_axis_name="s")


def _zero_accumulator(rows_v, agg_sh, s):
  def zrow(r, carry):
    def zcol(k, carry2):
      rows_v[r, pl.ds(k * 16, 16)] = jnp.zeros((16,), jnp.float32)
      return carry2
    return lax.fori_loop(0, D // 16, zcol, carry)
  lax.fori_loop(0, 128, zrow, 0)

  def zcopy(i, carry):
    pltpu.sync_copy(rows_v, agg_sh.at[pl.ds(s * RPT + i * 128, 128)])
    return carry
  lax.fori_loop(0, RPT // 128, zcopy, 0)


NQ = 5
QB = EB // NQ


def _scatter_edges(h_hbm, src_hbm, dst_hbm, src_v, dst_v, rows0, rows1,
                   agg_sh, sem0, sem1, wid):
  for q in range(NQ):
    pltpu.sync_copy(src_hbm.at[pl.ds(wid * EB + q * QB, QB)], src_v)
    pltpu.sync_copy(dst_hbm.at[pl.ds(wid * EB + q * QB, QB)], dst_v)
    pltpu.async_copy(h_hbm.at[src_v.at[0]], rows0, sem0)

    def pair(p, carry):
      j0 = 2 * p
      j1 = 2 * p + 1
      j2 = jnp.minimum(2 * p + 2, QB - 1)
      pltpu.make_async_copy(h_hbm.at[src_v.at[0]], rows0, sem0).wait()
      pltpu.async_copy(h_hbm.at[src_v.at[j1]], rows1, sem1)
      pltpu.sync_copy(rows0, agg_sh.at[dst_v.at[j0]], add=True)
      pltpu.make_async_copy(h_hbm.at[src_v.at[0]], rows1, sem1).wait()
      pltpu.async_copy(h_hbm.at[src_v.at[j2]], rows0, sem0)
      pltpu.sync_copy(rows1, agg_sh.at[dst_v.at[j1]], add=True)
      return carry
    lax.fori_loop(0, QB // 2, pair, 0)
    pltpu.make_async_copy(h_hbm.at[src_v.at[0]], rows0, sem0).wait()


@functools.partial(
    pl.kernel,
    out_type=jax.ShapeDtypeStruct((NC, AGG_ROWS, D), jnp.float32),
    mesh=_sc_mesh,
    scratch_types=[
        pltpu.VMEM((QB, 128), jnp.int32),
        pltpu.VMEM((QB, 128), jnp.int32),
        pltpu.VMEM((128, D), jnp.float32),
        pltpu.VMEM((128, D), jnp.float32),
        pltpu.VMEM_SHARED((AGG_ROWS, D), jnp.float32),
        pltpu.SemaphoreType.DMA,
        pltpu.SemaphoreType.DMA,
    ],
)
def _segsum_full(h_hbm, src_hbm, dst_hbm, out_hbm,
                 src_v, dst_v, rows0, rows1, agg_sh, sem0, sem1):
  c = lax.axis_index("c")
  s = lax.axis_index("s")
  wid = c * NS + s
  _zero_accumulator(rows0, agg_sh, s)
  plsc.subcore_barrier()
  _scatter_edges(h_hbm, src_hbm, dst_hbm, src_v, dst_v, rows0, rows1,
                 agg_sh, sem0, sem1, wid)
  plsc.subcore_barrier()
  pltpu.sync_copy(agg_sh.at[pl.ds(s * RPT, RPT)],
                  out_hbm.at[c, pl.ds(s * RPT, RPT)])


@functools.partial(
    pl.kernel,
    out_type=(jax.ShapeDtypeStruct((N_USERS, D), jnp.float32),
              jax.ShapeDtypeStruct((NC, N_USERS, D), jnp.float32)),
    mesh=_sc_mesh,
    scratch_types=[
        pltpu.VMEM((QB, 128), jnp.int32),
        pltpu.VMEM((QB, 128), jnp.int32),
        pltpu.VMEM((128, D), jnp.float32),
        pltpu.VMEM((128, D), jnp.float32),
        pltpu.VMEM((UPT,), jnp.int32),
        pltpu.VMEM_SHARED((AGG_ROWS, D), jnp.float32),
        pltpu.SemaphoreType.DMA,
        pltpu.SemaphoreType.DMA,
    ],
)
def _segsum_users(h_hbm, src_hbm, dst_hbm, uidx_hbm, uh_hbm, uagg_hbm,
                  src_v, dst_v, rows0, rows1, uidx_v, agg_sh, sem0, sem1):
  c = lax.axis_index("c")
  s = lax.axis_index("s")
  wid = c * NS + s
  _zero_accumulator(rows0, agg_sh, s)
  plsc.subcore_barrier()
  _scatter_edges(h_hbm, src_hbm, dst_hbm, src_v, dst_v, rows0, rows1,
                 agg_sh, sem0, sem1, wid)
  plsc.subcore_barrier()
  pltpu.sync_copy(uidx_hbm.at[pl.ds(s * UPT, UPT)], uidx_v)
  urows_v = rows0.at[pl.ds(0, UPT)]
  pltpu.async_copy(agg_sh.at[uidx_v], urows_v, sem0).wait()
  pltpu.sync_copy(urows_v, uagg_hbm.at[c, pl.ds(s * UPT, UPT)])

  @pl.when(c == 0)
  def _():
    pltpu.async_copy(h_hbm.at[uidx_v], urows_v, sem0).wait()
    pltpu.sync_copy(urows_v, uh_hbm.at[pl.ds(s * UPT, UPT)])


_NB = 10
_BR = N_NODES // _NB


def _linrelu_body(x_ref, w_ref, b_ref, o_ref):
  o_ref[...] = jnp.maximum(
      jnp.dot(x_ref[...], w_ref[...], preferred_element_type=jnp.float32)
      + b_ref[...], 0.0)


def _tc_linrelu(x, w, b):
  return pl.pallas_call(
      _linrelu_body,
      grid=(_NB,),
      in_specs=[
          pl.BlockSpec((_BR, D), lambda i: (i, 0)),
          pl.BlockSpec((D, D), lambda i: (0, 0)),
          pl.BlockSpec((1, D), lambda i: (0, 0)),
      ],
      out_specs=pl.BlockSpec((_BR, D), lambda i: (i, 0)),
      out_shape=jax.ShapeDtypeStruct((N_NODES, D), jnp.float32),
  )(x, w, b)


def _fuse_body(h_ref, a0_ref, a1_ref, w_ref, b_ref, o_ref):
  u = h_ref[...] + a0_ref[...] + a1_ref[...]
  o_ref[...] = jnp.maximum(
      jnp.dot(u, w_ref[...], preferred_element_type=jnp.float32)
      + b_ref[...], 0.0)


def _tc_fuse(h, a0, a1, w, b):
  return pl.pallas_call(
      _fuse_body,
      grid=(_NB,),
      in_specs=[
          pl.BlockSpec((_BR, D), lambda i: (i, 0)),
          pl.BlockSpec((_BR, D), lambda i: (i, 0)),
          pl.BlockSpec((_BR, D), lambda i: (i, 0)),
          pl.BlockSpec((D, D), lambda i: (0, 0)),
          pl.BlockSpec((1, D), lambda i: (0, 0)),
      ],
      out_specs=pl.BlockSpec((_BR, D), lambda i: (i, 0)),
      out_shape=jax.ShapeDtypeStruct((N_NODES, D), jnp.float32),
  )(h, a0, a1, w, b)


def _head_body(uh_ref, a0_ref, a1_ref, wg2_ref, bg2_ref, wpj_ref, bpj_ref,
               wp1a_ref, nlp_ref, wp1b_ref, bp1_ref, wp2_ref, bp2_ref,
               wp3_ref, bp3_ref, o_ref):
  f32 = jnp.float32
  u = uh_ref[...] + a0_ref[...] + a1_ref[...]
  h2 = jnp.maximum(
      jnp.dot(u, wg2_ref[...], preferred_element_type=f32) + bg2_ref[...], 0.0)
  emb = jnp.dot(h2, wpj_ref[...], preferred_element_type=f32) + bpj_ref[...]
  nz = jnp.dot(nlp_ref[...], wp1b_ref[...], preferred_element_type=f32)[0:1, :]
  z1 = jnp.maximum(
      jnp.dot(emb, wp1a_ref[...], preferred_element_type=f32)
      + nz + bp1_ref[...], 0.0)
  z2 = jnp.maximum(
      jnp.dot(z1, wp2_ref[...], preferred_element_type=f32) + bp2_ref[...],
      0.0)
  lg = jnp.dot(z2, wp3_ref[...], preferred_element_type=f32) + bp3_ref[...]
  o_ref[...] = jax.nn.sigmoid(lg)


def _tc_head(uh, a0, a1, wg2, bg2, wpj, bpj, wp1a, nlp_p, wp1b, bp1, wp2,
             bp2, wp3, bp3):
  return pl.pallas_call(
      _head_body,
      out_shape=jax.ShapeDtypeStruct((N_USERS, 128), jnp.float32),
  )(uh, a0, a1, wg2, bg2, wpj, bpj, wp1a, nlp_p, wp1b, bp1, wp2, bp2, wp3,
    bp3)


def kernel(x, nlp_features, edge_index, user_indices,
           W_in, b_in, W_g1, b_g1, W_g2, b_g2,
           W_proj, b_proj, W_p1, b_p1, W_p2, b_p2, W_p3, b_p3):
  f32 = jnp.float32
  src = edge_index[0].astype(jnp.int32)
  dst = edge_index[1].astype(jnp.int32)
  pad = E_PAD - N_EDGES
  src_p = jnp.concatenate(
      [src, jnp.zeros((pad,), jnp.int32)]).reshape(NW * EB, 128)
  dst_p = jnp.concatenate(
      [dst, jnp.full((pad,), TRASH_ROW, jnp.int32)]).reshape(NW * EB, 128)
  uidx = user_indices.astype(jnp.int32)

  h0 = _tc_linrelu(x, W_in, b_in.reshape(1, D))
  agg1 = _segsum_full(h0, src_p, dst_p)
  h1 = _tc_fuse(h0, agg1[0, :N_NODES], agg1[1, :N_NODES],
                W_g1, b_g1.reshape(1, D))
  uh1, uagg = _segsum_users(h1, src_p, dst_p, uidx)

  nlp_p = jnp.zeros((8, 896), f32).at[0, :NLP_DIM].set(nlp_features)
  wp1b = jnp.zeros((896, 256), f32).at[:NLP_DIM].set(W_p1[D:])
  wp3 = jnp.zeros((128, 128), f32).at[:, :1].set(W_p3)
  bp3 = jnp.zeros((1, 128), f32).at[0, 0].set(b_p3[0])

  out = _tc_head(uh1, uagg[0], uagg[1], W_g2, b_g2.reshape(1, D),
                 W_proj, b_proj.reshape(1, D), W_p1[:D], nlp_p, wp1b,
                 b_p1.reshape(1, 256), W_p2, b_p2.reshape(1, 128), wp3, bp3)
  return out[:, 0]


# speedup vs baseline: 9.2964x; 3.0235x over previous
import functools

import jax
import jax.numpy as jnp
from jax import lax
from jax.experimental import pallas as pl
from jax.experimental.pallas import tpu as pltpu
from jax.experimental.pallas import tpu_sc as plsc

N_NODES = 10000
D = 128
N_EDGES = 320000
N_USERS = 1024
NLP_DIM = 786

NC, NS = 2, 16
NW = NC * NS
EB = 80
E_PAD = NW * EB * 128
AGG_ROWS = 10240
RPT = AGG_ROWS // NS
TRASH_ROW = N_NODES
UPT = N_USERS // NS

_sc_mesh = plsc.VectorSubcoreMesh(core_axis_name="c", subcore_axis_name="s")


def _zero_accumulator(rows_v, agg_sh, s):
  def zrow(r, carry):
    def zcol(k, carry2):
      rows_v[r, pl.ds(k * 16, 16)] = jnp.zeros((16,), jnp.float32)
      return carry2
    return lax.fori_loop(0, D // 16, zcol, carry)
  lax.fori_loop(0, 128, zrow, 0)

  def zcopy(i, carry):
    pltpu.sync_copy(rows_v, agg_sh.at[pl.ds(s * RPT + i * 128, 128)])
    return carry
  lax.fori_loop(0, RPT // 128, zcopy, 0)


NQ = 5
QB = EB // NQ


def _scatter_edges(h_hbm, src_hbm, dst_hbm, src_v, dst_v, rows0, rows1,
                   agg_sh, sem0, sem1, wid):
  for q in range(NQ):
    pltpu.sync_copy(src_hbm.at[pl.ds(wid * EB + q * QB, QB)], src_v)
    pltpu.sync_copy(dst_hbm.at[pl.ds(wid * EB + q * QB, QB)], dst_v)
    pltpu.async_copy(h_hbm.at[src_v.at[0]], rows0, sem0)

    def pair(p, carry):
      j0 = 2 * p
      j1 = 2 * p + 1
      j2 = jnp.minimum(2 * p + 2, QB - 1)
      pltpu.make_async_copy(h_hbm.at[src_v.at[0]], rows0, sem0).wait()
      pltpu.async_copy(h_hbm.at[src_v.at[j1]], rows1, sem1)
      pltpu.sync_copy(rows0, agg_sh.at[dst_v.at[j0]], add=True)
      pltpu.make_async_copy(h_hbm.at[src_v.at[0]], rows1, sem1).wait()
      pltpu.async_copy(h_hbm.at[src_v.at[j2]], rows0, sem0)
      pltpu.sync_copy(rows1, agg_sh.at[dst_v.at[j1]], add=True)
      return carry
    lax.fori_loop(0, QB // 2, pair, 0)
    pltpu.make_async_copy(h_hbm.at[src_v.at[0]], rows0, sem0).wait()


@functools.partial(
    pl.kernel,
    out_type=jax.ShapeDtypeStruct((NC, AGG_ROWS, D), jnp.float32),
    mesh=_sc_mesh,
    scratch_types=[
        pltpu.VMEM((QB, 128), jnp.int32),
        pltpu.VMEM((QB, 128), jnp.int32),
        pltpu.VMEM((128, D), jnp.float32),
        pltpu.VMEM((128, D), jnp.float32),
        pltpu.VMEM_SHARED((AGG_ROWS, D), jnp.float32),
        pltpu.SemaphoreType.DMA,
        pltpu.SemaphoreType.DMA,
    ],
)
def _segsum_full(h_hbm, src_hbm, dst_hbm, out_hbm,
                 src_v, dst_v, rows0, rows1, agg_sh, sem0, sem1):
  c = lax.axis_index("c")
  s = lax.axis_index("s")
  wid = c * NS + s
  _zero_accumulator(rows0, agg_sh, s)
  plsc.subcore_barrier()
  _scatter_edges(h_hbm, src_hbm, dst_hbm, src_v, dst_v, rows0, rows1,
                 agg_sh, sem0, sem1, wid)
  plsc.subcore_barrier()
  pltpu.sync_copy(agg_sh.at[pl.ds(s * RPT, RPT)],
                  out_hbm.at[c, pl.ds(s * RPT, RPT)])


@functools.partial(
    pl.kernel,
    out_type=(jax.ShapeDtypeStruct((N_USERS, D), jnp.float32),
              jax.ShapeDtypeStruct((NC, N_USERS, D), jnp.float32)),
    mesh=_sc_mesh,
    scratch_types=[
        pltpu.VMEM((QB, 128), jnp.int32),
        pltpu.VMEM((QB, 128), jnp.int32),
        pltpu.VMEM((128, D), jnp.float32),
        pltpu.VMEM((128, D), jnp.float32),
        pltpu.VMEM((UPT,), jnp.int32),
        pltpu.VMEM_SHARED((AGG_ROWS, D), jnp.float32),
        pltpu.SemaphoreType.DMA,
        pltpu.SemaphoreType.DMA,
    ],
)
def _segsum_users(h_hbm, src_hbm, dst_hbm, uidx_hbm, uh_hbm, uagg_hbm,
                  src_v, dst_v, rows0, rows1, uidx_v, agg_sh, sem0, sem1):
  c = lax.axis_index("c")
  s = lax.axis_index("s")
  wid = c * NS + s
  _zero_accumulator(rows0, agg_sh, s)
  plsc.subcore_barrier()
  _scatter_edges(h_hbm, src_hbm, dst_hbm, src_v, dst_v, rows0, rows1,
                 agg_sh, sem0, sem1, wid)
  plsc.subcore_barrier()
  pltpu.sync_copy(uidx_hbm.at[pl.ds(s * UPT, UPT)], uidx_v)
  urows_v = rows0.at[pl.ds(0, UPT)]
  pltpu.async_copy(agg_sh.at[uidx_v], urows_v, sem0).wait()
  pltpu.sync_copy(urows_v, uagg_hbm.at[c, pl.ds(s * UPT, UPT)])

  @pl.when(c == 0)
  def _():
    pltpu.async_copy(h_hbm.at[uidx_v], urows_v, sem0).wait()
    pltpu.sync_copy(urows_v, uh_hbm.at[pl.ds(s * UPT, UPT)])


_NB = 10
_BR = N_NODES // _NB


def _linrelu_body(x_ref, w_ref, b_ref, o_ref):
  o_ref[...] = jnp.maximum(
      jnp.dot(x_ref[...], w_ref[...], preferred_element_type=jnp.float32)
      + b_ref[...], 0.0)


def _tc_linrelu(x, w, b):
  return pl.pallas_call(
      _linrelu_body,
      grid=(_NB,),
      in_specs=[
          pl.BlockSpec((_BR, D), lambda i: (i, 0)),
          pl.BlockSpec((D, D), lambda i: (0, 0)),
          pl.BlockSpec((1, D), lambda i: (0, 0)),
      ],
      out_specs=pl.BlockSpec((_BR, D), lambda i: (i, 0)),
      out_shape=jax.ShapeDtypeStruct((N_NODES, D), jnp.float32),
  )(x, w, b)


def _fuse_body(h_ref, a0_ref, a1_ref, w_ref, b_ref, o_ref):
  u = h_ref[...] + a0_ref[...] + a1_ref[...]
  o_ref[...] = jnp.maximum(
      jnp.dot(u, w_ref[...], preferred_element_type=jnp.float32)
      + b_ref[...], 0.0)


def _tc_fuse(h, a0, a1, w, b):
  return pl.pallas_call(
      _fuse_body,
      grid=(_NB,),
      in_specs=[
          pl.BlockSpec((_BR, D), lambda i: (i, 0)),
          pl.BlockSpec((_BR, D), lambda i: (i, 0)),
          pl.BlockSpec((_BR, D), lambda i: (i, 0)),
          pl.BlockSpec((D, D), lambda i: (0, 0)),
          pl.BlockSpec((1, D), lambda i: (0, 0)),
      ],
      out_specs=pl.BlockSpec((_BR, D), lambda i: (i, 0)),
      out_shape=jax.ShapeDtypeStruct((N_NODES, D), jnp.float32),
  )(h, a0, a1, w, b)


def _head_body(uh_ref, a0_ref, a1_ref, wg2_ref, bg2_ref, wpj_ref, bpj_ref,
               wp1a_ref, nlp_ref, wp1b_ref, bp1_ref, wp2_ref, bp2_ref,
               wp3_ref, bp3_ref, o_ref):
  f32 = jnp.float32
  u = uh_ref[...] + a0_ref[...] + a1_ref[...]
  h2 = jnp.maximum(
      jnp.dot(u, wg2_ref[...], preferred_element_type=f32) + bg2_ref[...], 0.0)
  emb = jnp.dot(h2, wpj_ref[...], preferred_element_type=f32) + bpj_ref[...]
  nz = jnp.dot(nlp_ref[...], wp1b_ref[...], preferred_element_type=f32)[0:1, :]
  z1 = jnp.maximum(
      jnp.dot(emb, wp1a_ref[...], preferred_element_type=f32)
      + nz + bp1_ref[...], 0.0)
  z2 = jnp.maximum(
      jnp.dot(z1, wp2_ref[...], preferred_element_type=f32) + bp2_ref[...],
      0.0)
  lg = jnp.dot(z2, wp3_ref[...], preferred_element_type=f32) + bp3_ref[...]
  o_ref[...] = jax.nn.sigmoid(lg)


def _tc_head(uh, a0, a1, wg2, bg2, wpj, bpj, wp1a, nlp_p, wp1b, bp1, wp2,
             bp2, wp3, bp3):
  return pl.pallas_call(
      _head_body,
      out_shape=jax.ShapeDtypeStruct((N_USERS, 128), jnp.float32),
  )(uh, a0, a1, wg2, bg2, wpj, bpj, wp1a, nlp_p, wp1b, bp1, wp2, bp2, wp3,
    bp3)


def kernel(x, nlp_features, edge_index, user_indices,
           W_in, b_in, W_g1, b_g1, W_g2, b_g2,
           W_proj, b_proj, W_p1, b_p1, W_p2, b_p2, W_p3, b_p3):
  f32 = jnp.float32
  src = edge_index[0].astype(jnp.int32)
  dst = edge_index[1].astype(jnp.int32)
  pad = E_PAD - N_EDGES
  pad_iota = jnp.arange(pad, dtype=jnp.int32)
  src_p = jnp.concatenate(
      [src, pad_iota % N_NODES]).reshape(NW * EB, 128)
  dst_p = jnp.concatenate(
      [dst, TRASH_ROW + pad_iota % (AGG_ROWS - N_NODES)]).reshape(NW * EB, 128)
  uidx = user_indices.astype(jnp.int32)

  h0 = _tc_linrelu(x, W_in, b_in.reshape(1, D))
  agg1 = _segsum_full(h0, src_p, dst_p)
  h1 = _tc_fuse(h0, agg1[0, :N_NODES], agg1[1, :N_NODES],
                W_g1, b_g1.reshape(1, D))
  uh1, uagg = _segsum_users(h1, src_p, dst_p, uidx)

  nlp_p = jnp.zeros((8, 896), f32).at[0, :NLP_DIM].set(nlp_features)
  wp1b = jnp.zeros((896, 256), f32).at[:NLP_DIM].set(W_p1[D:])
  wp3 = jnp.zeros((128, 128), f32).at[:, :1].set(W_p3)
  bp3 = jnp.zeros((1, 128), f32).at[0, 0].set(b_p3[0])

  out = _tc_head(uh1, uagg[0], uagg[1], W_g2, b_g2.reshape(1, D),
                 W_proj, b_proj.reshape(1, D), W_p1[:D], nlp_p, wp1b,
                 b_p1.reshape(1, 256), W_p2, b_p2.reshape(1, 128), wp3, bp3)
  return out[:, 0]


# speedup vs baseline: 9.4809x; 1.0198x over previous
import functools

import jax
import jax.numpy as jnp
from jax import lax
from jax.experimental import pallas as pl
from jax.experimental.pallas import tpu as pltpu
from jax.experimental.pallas import tpu_sc as plsc

N_NODES = 10000
D = 128
N_EDGES = 320000
N_USERS = 1024
NLP_DIM = 786

NC, NS = 2, 16
NW = NC * NS
EB = 80
E_PAD = NW * EB * 128
AGG_ROWS = 10240
RPT = AGG_ROWS // NS
TRASH_ROW = N_NODES
UPT = N_USERS // NS

_sc_mesh = plsc.VectorSubcoreMesh(core_axis_name="c", subcore_axis_name="s")


def _zero_accumulator(rows_v, agg_sh, s):
  def zrow(r, carry):
    def zcol(k, carry2):
      rows_v[r, pl.ds(k * 16, 16)] = jnp.zeros((16,), jnp.float32)
      return carry2
    return lax.fori_loop(0, D // 16, zcol, carry)
  lax.fori_loop(0, 128, zrow, 0)

  def zcopy(i, carry):
    pltpu.sync_copy(rows_v, agg_sh.at[pl.ds(s * RPT + i * 128, 128)])
    return carry
  lax.fori_loop(0, RPT // 128, zcopy, 0)


NQ = 5
QB = EB // NQ


def _scatter_edges(h_hbm, src_hbm, dst_hbm, src_v, dst_v, rows0, rows1,
                   agg_sh, sem0, sem1, wid):
  for q in range(NQ):
    pltpu.sync_copy(src_hbm.at[pl.ds(wid * EB + q * QB, QB)], src_v)
    pltpu.sync_copy(dst_hbm.at[pl.ds(wid * EB + q * QB, QB)], dst_v)
    pltpu.async_copy(h_hbm.at[src_v.at[0]], rows0, sem0)

    def pair(p, carry):
      j0 = 2 * p
      j1 = 2 * p + 1
      j2 = jnp.minimum(2 * p + 2, QB - 1)
      pltpu.make_async_copy(h_hbm.at[src_v.at[0]], rows0, sem0).wait()
      pltpu.async_copy(h_hbm.at[src_v.at[j1]], rows1, sem1)
      pltpu.sync_copy(rows0, agg_sh.at[dst_v.at[j0]], add=True)
      pltpu.make_async_copy(h_hbm.at[src_v.at[0]], rows1, sem1).wait()
      pltpu.async_copy(h_hbm.at[src_v.at[j2]], rows0, sem0)
      pltpu.sync_copy(rows1, agg_sh.at[dst_v.at[j1]], add=True)
      return carry
    lax.fori_loop(0, QB // 2, pair, 0)
    pltpu.make_async_copy(h_hbm.at[src_v.at[0]], rows0, sem0).wait()


@functools.partial(
    pl.kernel,
    out_type=jax.ShapeDtypeStruct((NC, AGG_ROWS, D), jnp.float32),
    mesh=_sc_mesh,
    scratch_types=[
        pltpu.VMEM((QB, 128), jnp.int32),
        pltpu.VMEM((QB, 128), jnp.int32),
        pltpu.VMEM((128, D), jnp.float32),
        pltpu.VMEM((128, D), jnp.float32),
        pltpu.VMEM_SHARED((AGG_ROWS, D), jnp.float32),
        pltpu.SemaphoreType.DMA,
        pltpu.SemaphoreType.DMA,
    ],
)
def _segsum_full(h_hbm, src_hbm, dst_hbm, out_hbm,
                 src_v, dst_v, rows0, rows1, agg_sh, sem0, sem1):
  c = lax.axis_index("c")
  s = lax.axis_index("s")
  wid = c * NS + s
  _zero_accumulator(rows0, agg_sh, s)
  plsc.subcore_barrier()
  _scatter_edges(h_hbm, src_hbm, dst_hbm, src_v, dst_v, rows0, rows1,
                 agg_sh, sem0, sem1, wid)
  plsc.subcore_barrier()
  pltpu.sync_copy(agg_sh.at[pl.ds(s * RPT, RPT)],
                  out_hbm.at[c, pl.ds(s * RPT, RPT)])


@functools.partial(
    pl.kernel,
    out_type=(jax.ShapeDtypeStruct((N_USERS, D), jnp.float32),
              jax.ShapeDtypeStruct((NC, N_USERS, D), jnp.float32)),
    mesh=_sc_mesh,
    scratch_types=[
        pltpu.VMEM((QB, 128), jnp.int32),
        pltpu.VMEM((QB, 128), jnp.int32),
        pltpu.VMEM((128, D), jnp.float32),
        pltpu.VMEM((128, D), jnp.float32),
        pltpu.VMEM((UPT,), jnp.int32),
        pltpu.VMEM_SHARED((AGG_ROWS, D), jnp.float32),
        pltpu.SemaphoreType.DMA,
        pltpu.SemaphoreType.DMA,
    ],
)
def _segsum_users(h_hbm, src_hbm, dst_hbm, uidx_hbm, uh_hbm, uagg_hbm,
                  src_v, dst_v, rows0, rows1, uidx_v, agg_sh, sem0, sem1):
  c = lax.axis_index("c")
  s = lax.axis_index("s")
  wid = c * NS + s
  _zero_accumulator(rows0, agg_sh, s)
  plsc.subcore_barrier()
  _scatter_edges(h_hbm, src_hbm, dst_hbm, src_v, dst_v, rows0, rows1,
                 agg_sh, sem0, sem1, wid)
  plsc.subcore_barrier()
  pltpu.sync_copy(uidx_hbm.at[pl.ds(s * UPT, UPT)], uidx_v)
  urows_v = rows0.at[pl.ds(0, UPT)]
  pltpu.async_copy(agg_sh.at[uidx_v], urows_v, sem0).wait()
  pltpu.sync_copy(urows_v, uagg_hbm.at[c, pl.ds(s * UPT, UPT)])

  @pl.when(c == 0)
  def _():
    pltpu.async_copy(h_hbm.at[uidx_v], urows_v, sem0).wait()
    pltpu.sync_copy(urows_v, uh_hbm.at[pl.ds(s * UPT, UPT)])


_NB = 10
_BR = N_NODES // _NB


def _linrelu_body(x_ref, w_ref, b_ref, o_ref):
  o_ref[...] = jnp.maximum(
      jnp.dot(x_ref[...], w_ref[...], preferred_element_type=jnp.float32)
      + b_ref[...], 0.0)


def _tc_linrelu(x, w, b):
  return pl.pallas_call(
      _linrelu_body,
      grid=(_NB,),
      in_specs=[
          pl.BlockSpec((_BR, D), lambda i: (i, 0)),
          pl.BlockSpec((D, D), lambda i: (0, 0)),
          pl.BlockSpec((1, D), lambda i: (0, 0)),
      ],
      out_specs=pl.BlockSpec((_BR, D), lambda i: (i, 0)),
      out_shape=jax.ShapeDtypeStruct((N_NODES, D), jnp.float32),
  )(x, w, b)


def _fuse_body(h_ref, a0_ref, a1_ref, w_ref, b_ref, o_ref):
  u = h_ref[...] + a0_ref[0] + a1_ref[0]
  o_ref[...] = jnp.maximum(
      jnp.dot(u, w_ref[...], preferred_element_type=jnp.float32)
      + b_ref[...], 0.0)


def _tc_fuse(h, agg, w, b):
  return pl.pallas_call(
      _fuse_body,
      grid=(_NB,),
      in_specs=[
          pl.BlockSpec((_BR, D), lambda i: (i, 0)),
          pl.BlockSpec((1, _BR, D), lambda i: (0, i, 0)),
          pl.BlockSpec((1, _BR, D), lambda i: (1, i, 0)),
          pl.BlockSpec((D, D), lambda i: (0, 0)),
          pl.BlockSpec((1, D), lambda i: (0, 0)),
      ],
      out_specs=pl.BlockSpec((_BR, D), lambda i: (i, 0)),
      out_shape=jax.ShapeDtypeStruct((N_NODES, D), jnp.float32),
  )(h, agg, agg, w, b)


def _head_body(uh_ref, a0_ref, a1_ref, wg2_ref, bg2_ref, wpj_ref, bpj_ref,
               wp1a_ref, nlp_ref, wp1b_ref, bp1_ref, wp2_ref, bp2_ref,
               wp3_ref, bp3_ref, o_ref):
  f32 = jnp.float32
  u = uh_ref[...] + a0_ref[...] + a1_ref[...]
  h2 = jnp.maximum(
      jnp.dot(u, wg2_ref[...], preferred_element_type=f32) + bg2_ref[...], 0.0)
  emb = jnp.dot(h2, wpj_ref[...], preferred_element_type=f32) + bpj_ref[...]
  nz = jnp.dot(nlp_ref[...], wp1b_ref[...], preferred_element_type=f32)[0:1, :]
  z1 = jnp.maximum(
      jnp.dot(emb, wp1a_ref[...], preferred_element_type=f32)
      + nz + bp1_ref[...], 0.0)
  z2 = jnp.maximum(
      jnp.dot(z1, wp2_ref[...], preferred_element_type=f32) + bp2_ref[...],
      0.0)
  lg = jnp.dot(z2, wp3_ref[...], preferred_element_type=f32) + bp3_ref[...]
  o_ref[...] = jax.nn.sigmoid(lg)


def _tc_head(uh, a0, a1, wg2, bg2, wpj, bpj, wp1a, nlp_p, wp1b, bp1, wp2,
             bp2, wp3, bp3):
  return pl.pallas_call(
      _head_body,
      out_shape=jax.ShapeDtypeStruct((N_USERS, 128), jnp.float32),
  )(uh, a0, a1, wg2, bg2, wpj, bpj, wp1a, nlp_p, wp1b, bp1, wp2, bp2, wp3,
    bp3)


def kernel(x, nlp_features, edge_index, user_indices,
           W_in, b_in, W_g1, b_g1, W_g2, b_g2,
           W_proj, b_proj, W_p1, b_p1, W_p2, b_p2, W_p3, b_p3):
  f32 = jnp.float32
  src = edge_index[0].astype(jnp.int32)
  dst = edge_index[1].astype(jnp.int32)
  pad = E_PAD - N_EDGES
  pad_iota = jnp.arange(pad, dtype=jnp.int32)
  src_p = jnp.concatenate(
      [src, pad_iota % N_NODES]).reshape(NW * EB, 128)
  dst_p = jnp.concatenate(
      [dst, TRASH_ROW + pad_iota % (AGG_ROWS - N_NODES)]).reshape(NW * EB, 128)
  uidx = user_indices.astype(jnp.int32)

  h0 = _tc_linrelu(x, W_in, b_in.reshape(1, D))
  agg1 = _segsum_full(h0, src_p, dst_p)
  h1 = _tc_fuse(h0, agg1, W_g1, b_g1.reshape(1, D))
  uh1, uagg = _segsum_users(h1, src_p, dst_p, uidx)

  nlp_p = jnp.zeros((8, 896), f32).at[0, :NLP_DIM].set(nlp_features)
  wp1b = jnp.zeros((896, 256), f32).at[:NLP_DIM].set(W_p1[D:])
  wp3 = jnp.zeros((128, 128), f32).at[:, :1].set(W_p3)
  bp3 = jnp.zeros((1, 128), f32).at[0, 0].set(b_p3[0])

  out = _tc_head(uh1, uagg[0], uagg[1], W_g2, b_g2.reshape(1, D),
                 W_proj, b_proj.reshape(1, D), W_p1[:D], nlp_p, wp1b,
                 b_p1.reshape(1, 256), W_p2, b_p2.reshape(1, 128), wp3, bp3)
  return out[:, 0]


# speedup vs baseline: 9.6799x; 1.0210x over previous
import functools

import jax
import jax.numpy as jnp
from jax import lax
from jax.experimental import pallas as pl
from jax.experimental.pallas import tpu as pltpu
from jax.experimental.pallas import tpu_sc as plsc

N_NODES = 10000
D = 128
N_EDGES = 320000
N_USERS = 1024
NLP_DIM = 786

NC, NS = 2, 16
NW = NC * NS
EB = 80
E_PAD = NW * EB * 128
AGG_ROWS = 10240
RPT = AGG_ROWS // NS
TRASH_ROW = N_NODES
UPT = N_USERS // NS

_sc_mesh = plsc.VectorSubcoreMesh(core_axis_name="c", subcore_axis_name="s")


def _zero_accumulator(rows_v, agg_sh, s):
  def zrow(r, carry):
    for k in range(D // 16):
      rows_v[r, pl.ds(k * 16, 16)] = jnp.zeros((16,), jnp.float32)
    return carry
  lax.fori_loop(0, 128, zrow, 0)

  def zcopy(i, carry):
    pltpu.sync_copy(rows_v, agg_sh.at[pl.ds(s * RPT + i * 128, 128)])
    return carry
  lax.fori_loop(0, RPT // 128, zcopy, 0)


NQ = 5
QB = EB // NQ


def _scatter_edges(h_hbm, src_hbm, dst_hbm, src_v, dst_v, rows0, rows1,
                   agg_sh, sem0, sem1, wid):
  for q in range(NQ):
    pltpu.sync_copy(src_hbm.at[pl.ds(wid * EB + q * QB, QB)], src_v)
    pltpu.sync_copy(dst_hbm.at[pl.ds(wid * EB + q * QB, QB)], dst_v)
    pltpu.async_copy(h_hbm.at[src_v.at[0]], rows0, sem0)

    def pair(p, carry):
      j0 = 2 * p
      j1 = 2 * p + 1
      j2 = 2 * p + 2
      pltpu.make_async_copy(h_hbm.at[src_v.at[0]], rows0, sem0).wait()
      pltpu.async_copy(h_hbm.at[src_v.at[j1]], rows1, sem1)
      pltpu.sync_copy(rows0, agg_sh.at[dst_v.at[j0]], add=True)
      pltpu.make_async_copy(h_hbm.at[src_v.at[0]], rows1, sem1).wait()
      pltpu.async_copy(h_hbm.at[src_v.at[j2]], rows0, sem0)
      pltpu.sync_copy(rows1, agg_sh.at[dst_v.at[j1]], add=True)
      return carry
    lax.fori_loop(0, QB // 2 - 1, pair, 0)
    pltpu.make_async_copy(h_hbm.at[src_v.at[0]], rows0, sem0).wait()
    pltpu.async_copy(h_hbm.at[src_v.at[QB - 1]], rows1, sem1)
    pltpu.sync_copy(rows0, agg_sh.at[dst_v.at[QB - 2]], add=True)
    pltpu.make_async_copy(h_hbm.at[src_v.at[0]], rows1, sem1).wait()
    pltpu.sync_copy(rows1, agg_sh.at[dst_v.at[QB - 1]], add=True)


@functools.partial(
    pl.kernel,
    out_type=jax.ShapeDtypeStruct((NC, AGG_ROWS, D), jnp.float32),
    mesh=_sc_mesh,
    scratch_types=[
        pltpu.VMEM((QB, 128), jnp.int32),
        pltpu.VMEM((QB, 128), jnp.int32),
        pltpu.VMEM((128, D), jnp.float32),
        pltpu.VMEM((128, D), jnp.float32),
        pltpu.VMEM_SHARED((AGG_ROWS, D), jnp.float32),
        pltpu.SemaphoreType.DMA,
        pltpu.SemaphoreType.DMA,
    ],
)
def _segsum_full(h_hbm, src_hbm, dst_hbm, out_hbm,
                 src_v, dst_v, rows0, rows1, agg_sh, sem0, sem1):
  c = lax.axis_index("c")
  s = lax.axis_index("s")
  wid = c * NS + s
  _zero_accumulator(rows0, agg_sh, s)
  plsc.subcore_barrier()
  _scatter_edges(h_hbm, src_hbm, dst_hbm, src_v, dst_v, rows0, rows1,
                 agg_sh, sem0, sem1, wid)
  plsc.subcore_barrier()
  pltpu.sync_copy(agg_sh.at[pl.ds(s * RPT, RPT)],
                  out_hbm.at[c, pl.ds(s * RPT, RPT)])


@functools.partial(
    pl.kernel,
    out_type=(jax.ShapeDtypeStruct((N_USERS, D), jnp.float32),
              jax.ShapeDtypeStruct((NC, N_USERS, D), jnp.float32)),
    mesh=_sc_mesh,
    scratch_types=[
        pltpu.VMEM((QB, 128), jnp.int32),
        pltpu.VMEM((QB, 128), jnp.int32),
        pltpu.VMEM((128, D), jnp.float32),
        pltpu.VMEM((128, D), jnp.float32),
        pltpu.VMEM((UPT,), jnp.int32),
        pltpu.VMEM_SHARED((AGG_ROWS, D), jnp.float32),
        pltpu.SemaphoreType.DMA,
        pltpu.SemaphoreType.DMA,
    ],
)
def _segsum_users(h_hbm, src_hbm, dst_hbm, uidx_hbm, uh_hbm, uagg_hbm,
                  src_v, dst_v, rows0, rows1, uidx_v, agg_sh, sem0, sem1):
  c = lax.axis_index("c")
  s = lax.axis_index("s")
  wid = c * NS + s
  _zero_accumulator(rows0, agg_sh, s)
  plsc.subcore_barrier()
  _scatter_edges(h_hbm, src_hbm, dst_hbm, src_v, dst_v, rows0, rows1,
                 agg_sh, sem0, sem1, wid)
  plsc.subcore_barrier()
  pltpu.sync_copy(uidx_hbm.at[pl.ds(s * UPT, UPT)], uidx_v)
  urows_v = rows0.at[pl.ds(0, UPT)]
  pltpu.async_copy(agg_sh.at[uidx_v], urows_v, sem0).wait()
  pltpu.sync_copy(urows_v, uagg_hbm.at[c, pl.ds(s * UPT, UPT)])

  @pl.when(c == 0)
  def _():
    pltpu.async_copy(h_hbm.at[uidx_v], urows_v, sem0).wait()
    pltpu.sync_copy(urows_v, uh_hbm.at[pl.ds(s * UPT, UPT)])


_NB = 10
_BR = N_NODES // _NB


def _linrelu_body(x_ref, w_ref, b_ref, o_ref):
  o_ref[...] = jnp.maximum(
      jnp.dot(x_ref[...], w_ref[...], preferred_element_type=jnp.float32)
      + b_ref[...], 0.0)


def _tc_linrelu(x, w, b):
  return pl.pallas_call(
      _linrelu_body,
      grid=(_NB,),
      in_specs=[
          pl.BlockSpec((_BR, D), lambda i: (i, 0)),
          pl.BlockSpec((D, D), lambda i: (0, 0)),
          pl.BlockSpec((1, D), lambda i: (0, 0)),
      ],
      out_specs=pl.BlockSpec((_BR, D), lambda i: (i, 0)),
      out_shape=jax.ShapeDtypeStruct((N_NODES, D), jnp.float32),
  )(x, w, b)


def _fuse_body(h_ref, a0_ref, a1_ref, w_ref, b_ref, o_ref):
  u = h_ref[...] + a0_ref[0] + a1_ref[0]
  o_ref[...] = jnp.maximum(
      jnp.dot(u, w_ref[...], preferred_element_type=jnp.float32)
      + b_ref[...], 0.0)


def _tc_fuse(h, agg, w, b):
  return pl.pallas_call(
      _fuse_body,
      grid=(_NB,),
      in_specs=[
          pl.BlockSpec((_BR, D), lambda i: (i, 0)),
          pl.BlockSpec((1, _BR, D), lambda i: (0, i, 0)),
          pl.BlockSpec((1, _BR, D), lambda i: (1, i, 0)),
          pl.BlockSpec((D, D), lambda i: (0, 0)),
          pl.BlockSpec((1, D), lambda i: (0, 0)),
      ],
      out_specs=pl.BlockSpec((_BR, D), lambda i: (i, 0)),
      out_shape=jax.ShapeDtypeStruct((N_NODES, D), jnp.float32),
  )(h, agg, agg, w, b)


def _head_body(uh_ref, a0_ref, a1_ref, wg2_ref, bg2_ref, wpj_ref, bpj_ref,
               wp1a_ref, nlp_ref, wp1b_ref, bp1_ref, wp2_ref, bp2_ref,
               wp3_ref, bp3_ref, o_ref):
  f32 = jnp.float32
  u = uh_ref[...] + a0_ref[...] + a1_ref[...]
  h2 = jnp.maximum(
      jnp.dot(u, wg2_ref[...], preferred_element_type=f32) + bg2_ref[...], 0.0)
  emb = jnp.dot(h2, wpj_ref[...], preferred_element_type=f32) + bpj_ref[...]
  nz = jnp.dot(nlp_ref[...], wp1b_ref[...], preferred_element_type=f32)[0:1, :]
  z1 = jnp.maximum(
      jnp.dot(emb, wp1a_ref[...], preferred_element_type=f32)
      + nz + bp1_ref[...], 0.0)
  z2 = jnp.maximum(
      jnp.dot(z1, wp2_ref[...], preferred_element_type=f32) + bp2_ref[...],
      0.0)
  lg = jnp.dot(z2, wp3_ref[...], preferred_element_type=f32) + bp3_ref[...]
  o_ref[...] = jax.nn.sigmoid(lg)


def _tc_head(uh, a0, a1, wg2, bg2, wpj, bpj, wp1a, nlp_p, wp1b, bp1, wp2,
             bp2, wp3, bp3):
  return pl.pallas_call(
      _head_body,
      out_shape=jax.ShapeDtypeStruct((N_USERS, 128), jnp.float32),
  )(uh, a0, a1, wg2, bg2, wpj, bpj, wp1a, nlp_p, wp1b, bp1, wp2, bp2, wp3,
    bp3)


def kernel(x, nlp_features, edge_index, user_indices,
           W_in, b_in, W_g1, b_g1, W_g2, b_g2,
           W_proj, b_proj, W_p1, b_p1, W_p2, b_p2, W_p3, b_p3):
  f32 = jnp.float32
  src = edge_index[0].astype(jnp.int32)
  dst = edge_index[1].astype(jnp.int32)
  pad = E_PAD - N_EDGES
  pad_iota = jnp.arange(pad, dtype=jnp.int32)
  src_p = jnp.concatenate(
      [src, pad_iota % N_NODES]).reshape(NW * EB, 128)
  dst_p = jnp.concatenate(
      [dst, TRASH_ROW + pad_iota % (AGG_ROWS - N_NODES)]).reshape(NW * EB, 128)
  uidx = user_indices.astype(jnp.int32)

  h0 = _tc_linrelu(x, W_in, b_in.reshape(1, D))
  agg1 = _segsum_full(h0, src_p, dst_p)
  h1 = _tc_fuse(h0, agg1, W_g1, b_g1.reshape(1, D))
  uh1, uagg = _segsum_users(h1, src_p, dst_p, uidx)

  nlp_p = jnp.zeros((8, 896), f32).at[0, :NLP_DIM].set(nlp_features)
  wp1b = jnp.zeros((896, 256), f32).at[:NLP_DIM].set(W_p1[D:])
  wp3 = jnp.zeros((128, 128), f32).at[:, :1].set(W_p3)
  bp3 = jnp.zeros((1, 128), f32).at[0, 0].set(b_p3[0])

  out = _tc_head(uh1, uagg[0], uagg[1], W_g2, b_g2.reshape(1, D),
                 W_proj, b_proj.reshape(1, D), W_p1[:D], nlp_p, wp1b,
                 b_p1.reshape(1, 256), W_p2, b_p2.reshape(1, 128), wp3, bp3)
  return out[:, 0]


# speedup vs baseline: 10.1200x; 1.0455x over previous
import functools

import jax
import jax.numpy as jnp
from jax import lax
from jax.experimental import pallas as pl
from jax.experimental.pallas import tpu as pltpu
from jax.experimental.pallas import tpu_sc as plsc

N_NODES = 10000
D = 128
N_EDGES = 320000
N_USERS = 1024
NLP_DIM = 786

NC, NS = 2, 16
NW = NC * NS
EB = 80
E_PAD = NW * EB * 128
AGG_ROWS = 10112
RPT = AGG_ROWS // NS
TRASH_ROW = N_NODES
UPT = N_USERS // NS

_sc_mesh = plsc.VectorSubcoreMesh(core_axis_name="c", subcore_axis_name="s")


def _zero_accumulator(rows_v, agg_sh, s):
  def zrow(r, carry):
    for k in range(D // 16):
      rows_v[r, pl.ds(k * 16, 16)] = jnp.zeros((16,), jnp.float32)
    return carry
  lax.fori_loop(0, 128, zrow, 0)

  def zcopy(i, carry):
    pltpu.sync_copy(rows_v, agg_sh.at[pl.ds(s * RPT + i * 128, 128)])
    return carry
  lax.fori_loop(0, RPT // 128, zcopy, 0)
  rem = RPT % 128
  if rem:
    pltpu.sync_copy(rows_v.at[pl.ds(0, rem)],
                    agg_sh.at[pl.ds(s * RPT + RPT - rem, rem)])


NQ = 5
QB = EB // NQ


def _scatter_edges(h_hbm, sd_hbm, idx_a, idx_b, rows0, rows1,
                   agg_sh, sem0, sem1, sem_i, wid):
  bufs = (idx_a, idx_b)
  pltpu.sync_copy(sd_hbm.at[:, pl.ds(wid * EB, QB)], idx_a)
  pltpu.async_copy(h_hbm.at[idx_a.at[0, 0]], rows0, sem0)
  for q in range(NQ):
    cur = bufs[q % 2]
    nxt = bufs[(q + 1) % 2]
    if q + 1 < NQ:
      pltpu.async_copy(sd_hbm.at[:, pl.ds(wid * EB + (q + 1) * QB, QB)],
                       nxt, sem_i)

    def pair(p, carry, cur=cur):
      j0 = 2 * p
      j1 = 2 * p + 1
      j2 = 2 * p + 2
      pltpu.make_async_copy(h_hbm.at[cur.at[0, 0]], rows0, sem0).wait()
      pltpu.async_copy(h_hbm.at[cur.at[0, j1]], rows1, sem1)
      pltpu.sync_copy(rows0, agg_sh.at[cur.at[1, j0]], add=True)
      pltpu.make_async_copy(h_hbm.at[cur.at[0, 0]], rows1, sem1).wait()
      pltpu.async_copy(h_hbm.at[cur.at[0, j2]], rows0, sem0)
      pltpu.sync_copy(rows1, agg_sh.at[cur.at[1, j1]], add=True)
      return carry
    lax.fori_loop(0, QB // 2 - 1, pair, 0)
    pltpu.make_async_copy(h_hbm.at[cur.at[0, 0]], rows0, sem0).wait()
    pltpu.async_copy(h_hbm.at[cur.at[0, QB - 1]], rows1, sem1)
    pltpu.sync_copy(rows0, agg_sh.at[cur.at[1, QB - 2]], add=True)
    pltpu.make_async_copy(h_hbm.at[cur.at[0, 0]], rows1, sem1).wait()
    if q + 1 < NQ:
      pltpu.make_async_copy(sd_hbm.at[:, pl.ds(0, QB)], nxt, sem_i).wait()
      pltpu.async_copy(h_hbm.at[nxt.at[0, 0]], rows0, sem0)
    pltpu.sync_copy(rows1, agg_sh.at[cur.at[1, QB - 1]], add=True)


@functools.partial(
    pl.kernel,
    out_type=jax.ShapeDtypeStruct((NC, AGG_ROWS, D), jnp.float32),
    mesh=_sc_mesh,
    scratch_types=[
        pltpu.VMEM((2, QB, 128), jnp.int32),
        pltpu.VMEM((2, QB, 128), jnp.int32),
        pltpu.VMEM((128, D), jnp.float32),
        pltpu.VMEM((128, D), jnp.float32),
        pltpu.VMEM_SHARED((AGG_ROWS, D), jnp.float32),
        pltpu.SemaphoreType.DMA,
        pltpu.SemaphoreType.DMA,
        pltpu.SemaphoreType.DMA,
    ],
)
def _segsum_full(h_hbm, sd_hbm, out_hbm,
                 idx_a, idx_b, rows0, rows1, agg_sh, sem0, sem1, sem_i):
  c = lax.axis_index("c")
  s = lax.axis_index("s")
  wid = c * NS + s
  _zero_accumulator(rows0, agg_sh, s)
  plsc.subcore_barrier()
  _scatter_edges(h_hbm, sd_hbm, idx_a, idx_b, rows0, rows1,
                 agg_sh, sem0, sem1, sem_i, wid)
  plsc.subcore_barrier()
  pltpu.sync_copy(agg_sh.at[pl.ds(s * RPT, RPT)],
                  out_hbm.at[c, pl.ds(s * RPT, RPT)])


@functools.partial(
    pl.kernel,
    out_type=(jax.ShapeDtypeStruct((N_USERS, D), jnp.float32),
              jax.ShapeDtypeStruct((NC, N_USERS, D), jnp.float32)),
    mesh=_sc_mesh,
    scratch_types=[
        pltpu.VMEM((2, QB, 128), jnp.int32),
        pltpu.VMEM((2, QB, 128), jnp.int32),
        pltpu.VMEM((128, D), jnp.float32),
        pltpu.VMEM((128, D), jnp.float32),
        pltpu.VMEM((UPT,), jnp.int32),
        pltpu.VMEM_SHARED((AGG_ROWS, D), jnp.float32),
        pltpu.SemaphoreType.DMA,
        pltpu.SemaphoreType.DMA,
        pltpu.SemaphoreType.DMA,
    ],
)
def _segsum_users(h_hbm, sd_hbm, uidx_hbm, uh_hbm, uagg_hbm,
                  idx_a, idx_b, rows0, rows1, uidx_v, agg_sh,
                  sem0, sem1, sem_i):
  c = lax.axis_index("c")
  s = lax.axis_index("s")
  wid = c * NS + s
  _zero_accumulator(rows0, agg_sh, s)
  plsc.subcore_barrier()
  _scatter_edges(h_hbm, sd_hbm, idx_a, idx_b, rows0, rows1,
                 agg_sh, sem0, sem1, sem_i, wid)
  plsc.subcore_barrier()
  pltpu.sync_copy(uidx_hbm.at[pl.ds(s * UPT, UPT)], uidx_v)
  urows_v = rows0.at[pl.ds(0, UPT)]
  pltpu.async_copy(agg_sh.at[uidx_v], urows_v, sem0).wait()
  pltpu.sync_copy(urows_v, uagg_hbm.at[c, pl.ds(s * UPT, UPT)])

  @pl.when(c == 0)
  def _():
    pltpu.async_copy(h_hbm.at[uidx_v], urows_v, sem0).wait()
    pltpu.sync_copy(urows_v, uh_hbm.at[pl.ds(s * UPT, UPT)])


_NB = 10
_BR = N_NODES // _NB


def _linrelu_body(x_ref, w_ref, b_ref, o_ref):
  o_ref[...] = jnp.maximum(
      jnp.dot(x_ref[...], w_ref[...], preferred_element_type=jnp.float32)
      + b_ref[...], 0.0)


def _tc_linrelu(x, w, b):
  return pl.pallas_call(
      _linrelu_body,
      grid=(_NB,),
      in_specs=[
          pl.BlockSpec((_BR, D), lambda i: (i, 0)),
          pl.BlockSpec((D, D), lambda i: (0, 0)),
          pl.BlockSpec((1, D), lambda i: (0, 0)),
      ],
      out_specs=pl.BlockSpec((_BR, D), lambda i: (i, 0)),
      out_shape=jax.ShapeDtypeStruct((N_NODES, D), jnp.float32),
  )(x, w, b)


def _fuse_body(h_ref, a0_ref, a1_ref, w_ref, b_ref, o_ref):
  u = h_ref[...] + a0_ref[0] + a1_ref[0]
  o_ref[...] = jnp.maximum(
      jnp.dot(u, w_ref[...], preferred_element_type=jnp.float32)
      + b_ref[...], 0.0)


def _tc_fuse(h, agg, w, b):
  return pl.pallas_call(
      _fuse_body,
      grid=(_NB,),
      in_specs=[
          pl.BlockSpec((_BR, D), lambda i: (i, 0)),
          pl.BlockSpec((1, _BR, D), lambda i: (0, i, 0)),
          pl.BlockSpec((1, _BR, D), lambda i: (1, i, 0)),
          pl.BlockSpec((D, D), lambda i: (0, 0)),
          pl.BlockSpec((1, D), lambda i: (0, 0)),
      ],
      out_specs=pl.BlockSpec((_BR, D), lambda i: (i, 0)),
      out_shape=jax.ShapeDtypeStruct((N_NODES, D), jnp.float32),
  )(h, agg, agg, w, b)


def _head_body(uh_ref, a0_ref, a1_ref, wg2_ref, bg2_ref, wpj_ref, bpj_ref,
               wp1a_ref, nlp_ref, wp1b_ref, bp1_ref, wp2_ref, bp2_ref,
               wp3_ref, bp3_ref, o_ref):
  f32 = jnp.float32
  u = uh_ref[...] + a0_ref[...] + a1_ref[...]
  h2 = jnp.maximum(
      jnp.dot(u, wg2_ref[...], preferred_element_type=f32) + bg2_ref[...], 0.0)
  emb = jnp.dot(h2, wpj_ref[...], preferred_element_type=f32) + bpj_ref[...]
  nz = jnp.dot(nlp_ref[...], wp1b_ref[...], preferred_element_type=f32)[0:1, :]
  z1 = jnp.maximum(
      jnp.dot(emb, wp1a_ref[...], preferred_element_type=f32)
      + nz + bp1_ref[...], 0.0)
  z2 = jnp.maximum(
      jnp.dot(z1, wp2_ref[...], preferred_element_type=f32) + bp2_ref[...],
      0.0)
  lg = jnp.dot(z2, wp3_ref[...], preferred_element_type=f32) + bp3_ref[...]
  o_ref[...] = jax.nn.sigmoid(lg)


def _tc_head(uh, a0, a1, wg2, bg2, wpj, bpj, wp1a, nlp_p, wp1b, bp1, wp2,
             bp2, wp3, bp3):
  return pl.pallas_call(
      _head_body,
      out_shape=jax.ShapeDtypeStruct((N_USERS, 128), jnp.float32),
  )(uh, a0, a1, wg2, bg2, wpj, bpj, wp1a, nlp_p, wp1b, bp1, wp2, bp2, wp3,
    bp3)


def kernel(x, nlp_features, edge_index, user_indices,
           W_in, b_in, W_g1, b_g1, W_g2, b_g2,
           W_proj, b_proj, W_p1, b_p1, W_p2, b_p2, W_p3, b_p3):
  f32 = jnp.float32
  src = edge_index[0].astype(jnp.int32)
  dst = edge_index[1].astype(jnp.int32)
  pad = E_PAD - N_EDGES
  pad_iota = jnp.arange(pad, dtype=jnp.int32)
  src_p = jnp.concatenate(
      [src, pad_iota % N_NODES]).reshape(NW * EB, 128)
  dst_p = jnp.concatenate(
      [dst, TRASH_ROW + pad_iota % (AGG_ROWS - N_NODES)]).reshape(NW * EB, 128)
  sd = jnp.stack([src_p, dst_p])
  uidx = user_indices.astype(jnp.int32)

  h0 = _tc_linrelu(x, W_in, b_in.reshape(1, D))
  agg1 = _segsum_full(h0, sd)
  h1 = _tc_fuse(h0, agg1, W_g1, b_g1.reshape(1, D))
  uh1, uagg = _segsum_users(h1, sd, uidx)

  nlp_p = jnp.zeros((8, 896), f32).at[0, :NLP_DIM].set(nlp_features)
  wp1b = jnp.zeros((896, 256), f32).at[:NLP_DIM].set(W_p1[D:])
  wp3 = jnp.zeros((128, 128), f32).at[:, :1].set(W_p3)
  bp3 = jnp.zeros((1, 128), f32).at[0, 0].set(b_p3[0])

  out = _tc_head(uh1, uagg[0], uagg[1], W_g2, b_g2.reshape(1, D),
                 W_proj, b_proj.reshape(1, D), W_p1[:D], nlp_p, wp1b,
                 b_p1.reshape(1, 256), W_p2, b_p2.reshape(1, 128), wp3, bp3)
  return out[:, 0]


# speedup vs baseline: 10.3024x; 1.0180x over previous
import functools

import jax
import jax.numpy as jnp
from jax import lax
from jax.experimental import pallas as pl
from jax.experimental.pallas import tpu as pltpu
from jax.experimental.pallas import tpu_sc as plsc

N_NODES = 10000
D = 128
N_EDGES = 320000
N_USERS = 1024
NLP_DIM = 786

NC, NS = 2, 16
NW = NC * NS
EB = 80
E_PAD = NW * EB * 128
AGG_ROWS = 10112
RPT = AGG_ROWS // NS
TRASH_ROW = N_NODES
UPT = N_USERS // NS

_sc_mesh = plsc.VectorSubcoreMesh(core_axis_name="c", subcore_axis_name="s")


def _zero_accumulator(rows_v, agg_sh, s):
  def zrow(r, carry):
    for k in range(D // 16):
      rows_v[r, pl.ds(k * 16, 16)] = jnp.zeros((16,), jnp.float32)
    return carry
  lax.fori_loop(0, 128, zrow, 0)

  def zcopy(i, carry):
    pltpu.sync_copy(rows_v, agg_sh.at[pl.ds(s * RPT + i * 128, 128)])
    return carry
  lax.fori_loop(0, RPT // 128, zcopy, 0)
  rem = RPT % 128
  if rem:
    pltpu.sync_copy(rows_v.at[pl.ds(0, rem)],
                    agg_sh.at[pl.ds(s * RPT + RPT - rem, rem)])


NQ = 5
QB = EB // NQ


def _scatter_edges(h_hbm, sd_hbm, idx_a, idx_b, rows0, rows1,
                   agg_sh, sem0, sem1, sem_i, wid):
  bufs = (idx_a, idx_b)
  pltpu.sync_copy(sd_hbm.at[:, pl.ds(wid * EB, QB)], idx_a)
  pltpu.async_copy(h_hbm.at[idx_a.at[0, 0]], rows0, sem0)
  for q in range(NQ):
    cur = bufs[q % 2]
    nxt = bufs[(q + 1) % 2]
    if q + 1 < NQ:
      pltpu.async_copy(sd_hbm.at[:, pl.ds(wid * EB + (q + 1) * QB, QB)],
                       nxt, sem_i)

    def pair(p, carry, cur=cur):
      j0 = 2 * p
      j1 = 2 * p + 1
      j2 = 2 * p + 2
      pltpu.make_async_copy(h_hbm.at[cur.at[0, 0]], rows0, sem0).wait()
      pltpu.async_copy(h_hbm.at[cur.at[0, j1]], rows1, sem1)
      pltpu.sync_copy(rows0, agg_sh.at[cur.at[1, j0]], add=True)
      pltpu.make_async_copy(h_hbm.at[cur.at[0, 0]], rows1, sem1).wait()
      pltpu.async_copy(h_hbm.at[cur.at[0, j2]], rows0, sem0)
      pltpu.sync_copy(rows1, agg_sh.at[cur.at[1, j1]], add=True)
      return carry
    lax.fori_loop(0, QB // 2 - 1, pair, 0)
    pltpu.make_async_copy(h_hbm.at[cur.at[0, 0]], rows0, sem0).wait()
    pltpu.async_copy(h_hbm.at[cur.at[0, QB - 1]], rows1, sem1)
    pltpu.sync_copy(rows0, agg_sh.at[cur.at[1, QB - 2]], add=True)
    pltpu.make_async_copy(h_hbm.at[cur.at[0, 0]], rows1, sem1).wait()
    if q + 1 < NQ:
      pltpu.make_async_copy(sd_hbm.at[:, pl.ds(0, QB)], nxt, sem_i).wait()
      pltpu.async_copy(h_hbm.at[nxt.at[0, 0]], rows0, sem0)
    pltpu.sync_copy(rows1, agg_sh.at[cur.at[1, QB - 1]], add=True)


@functools.partial(
    pl.kernel,
    out_type=jax.ShapeDtypeStruct((NC, AGG_ROWS, D), jnp.float32),
    mesh=_sc_mesh,
    scratch_types=[
        pltpu.VMEM((2, QB, 128), jnp.int32),
        pltpu.VMEM((2, QB, 128), jnp.int32),
        pltpu.VMEM((128, D), jnp.float32),
        pltpu.VMEM((128, D), jnp.float32),
        pltpu.VMEM_SHARED((AGG_ROWS, D), jnp.float32),
        pltpu.SemaphoreType.DMA,
        pltpu.SemaphoreType.DMA,
        pltpu.SemaphoreType.DMA,
    ],
)
def _segsum_full(h_hbm, sd_hbm, out_hbm,
                 idx_a, idx_b, rows0, rows1, agg_sh, sem0, sem1, sem_i):
  c = lax.axis_index("c")
  s = lax.axis_index("s")
  wid = c * NS + s
  _zero_accumulator(rows0, agg_sh, s)
  plsc.subcore_barrier()
  _scatter_edges(h_hbm, sd_hbm, idx_a, idx_b, rows0, rows1,
                 agg_sh, sem0, sem1, sem_i, wid)
  plsc.subcore_barrier()
  pltpu.sync_copy(agg_sh.at[pl.ds(s * RPT, RPT)],
                  out_hbm.at[c, pl.ds(s * RPT, RPT)])


@functools.partial(
    pl.kernel,
    out_type=(jax.ShapeDtypeStruct((N_USERS, D), jnp.float32),
              jax.ShapeDtypeStruct((NC, N_USERS, D), jnp.float32)),
    mesh=_sc_mesh,
    scratch_types=[
        pltpu.VMEM((2, QB, 128), jnp.int32),
        pltpu.VMEM((2, QB, 128), jnp.int32),
        pltpu.VMEM((128, D), jnp.float32),
        pltpu.VMEM((128, D), jnp.float32),
        pltpu.VMEM((UPT,), jnp.int32),
        pltpu.VMEM_SHARED((AGG_ROWS, D), jnp.float32),
        pltpu.SemaphoreType.DMA,
        pltpu.SemaphoreType.DMA,
        pltpu.SemaphoreType.DMA,
    ],
)
def _segsum_users(h_hbm, sd_hbm, uidx_hbm, uh_hbm, uagg_hbm,
                  idx_a, idx_b, rows0, rows1, uidx_v, agg_sh,
                  sem0, sem1, sem_i):
  c = lax.axis_index("c")
  s = lax.axis_index("s")
  wid = c * NS + s
  _zero_accumulator(rows0, agg_sh, s)
  plsc.subcore_barrier()
  _scatter_edges(h_hbm, sd_hbm, idx_a, idx_b, rows0, rows1,
                 agg_sh, sem0, sem1, sem_i, wid)
  plsc.subcore_barrier()
  pltpu.sync_copy(uidx_hbm.at[pl.ds(s * UPT, UPT)], uidx_v)
  urows_v = rows0.at[pl.ds(0, UPT)]
  pltpu.async_copy(agg_sh.at[uidx_v], urows_v, sem0).wait()
  pltpu.sync_copy(urows_v, uagg_hbm.at[c, pl.ds(s * UPT, UPT)])

  @pl.when(c == 0)
  def _():
    pltpu.async_copy(h_hbm.at[uidx_v], urows_v, sem0).wait()
    pltpu.sync_copy(urows_v, uh_hbm.at[pl.ds(s * UPT, UPT)])


_NB = 5
_BR = N_NODES // _NB


def _linrelu_body(x_ref, w_ref, b_ref, o_ref):
  o_ref[...] = jnp.maximum(
      jnp.dot(x_ref[...], w_ref[...], preferred_element_type=jnp.float32)
      + b_ref[...], 0.0)


def _tc_linrelu(x, w, b):
  return pl.pallas_call(
      _linrelu_body,
      grid=(_NB,),
      in_specs=[
          pl.BlockSpec((_BR, D), lambda i: (i, 0)),
          pl.BlockSpec((D, D), lambda i: (0, 0)),
          pl.BlockSpec((1, D), lambda i: (0, 0)),
      ],
      out_specs=pl.BlockSpec((_BR, D), lambda i: (i, 0)),
      out_shape=jax.ShapeDtypeStruct((N_NODES, D), jnp.float32),
  )(x, w, b)


def _fuse_body(h_ref, a0_ref, a1_ref, w_ref, b_ref, o_ref):
  u = h_ref[...] + a0_ref[0] + a1_ref[0]
  o_ref[...] = jnp.maximum(
      jnp.dot(u, w_ref[...], preferred_element_type=jnp.float32)
      + b_ref[...], 0.0)


def _tc_fuse(h, agg, w, b):
  return pl.pallas_call(
      _fuse_body,
      grid=(_NB,),
      in_specs=[
          pl.BlockSpec((_BR, D), lambda i: (i, 0)),
          pl.BlockSpec((1, _BR, D), lambda i: (0, i, 0)),
          pl.BlockSpec((1, _BR, D), lambda i: (1, i, 0)),
          pl.BlockSpec((D, D), lambda i: (0, 0)),
          pl.BlockSpec((1, D), lambda i: (0, 0)),
      ],
      out_specs=pl.BlockSpec((_BR, D), lambda i: (i, 0)),
      out_shape=jax.ShapeDtypeStruct((N_NODES, D), jnp.float32),
  )(h, agg, agg, w, b)


def _head_body(uh_ref, a0_ref, a1_ref, wg2_ref, bg2_ref, wpj_ref, bpj_ref,
               wp1a_ref, nlp_ref, wp1b_ref, bp1_ref, wp2_ref, bp2_ref,
               wp3_ref, bp3_ref, o_ref):
  f32 = jnp.float32
  u = uh_ref[...] + a0_ref[...] + a1_ref[...]
  h2 = jnp.maximum(
      jnp.dot(u, wg2_ref[...], preferred_element_type=f32) + bg2_ref[...], 0.0)
  emb = jnp.dot(h2, wpj_ref[...], preferred_element_type=f32) + bpj_ref[...]
  nz = jnp.dot(nlp_ref[...], wp1b_ref[...], preferred_element_type=f32)[0:1, :]
  z1 = jnp.maximum(
      jnp.dot(emb, wp1a_ref[...], preferred_element_type=f32)
      + nz + bp1_ref[...], 0.0)
  z2 = jnp.maximum(
      jnp.dot(z1, wp2_ref[...], preferred_element_type=f32) + bp2_ref[...],
      0.0)
  lg = jnp.dot(z2, wp3_ref[...], preferred_element_type=f32) + bp3_ref[...]
  o_ref[...] = jax.nn.sigmoid(lg)


def _tc_head(uh, a0, a1, wg2, bg2, wpj, bpj, wp1a, nlp_p, wp1b, bp1, wp2,
             bp2, wp3, bp3):
  return pl.pallas_call(
      _head_body,
      out_shape=jax.ShapeDtypeStruct((N_USERS, 128), jnp.float32),
  )(uh, a0, a1, wg2, bg2, wpj, bpj, wp1a, nlp_p, wp1b, bp1, wp2, bp2, wp3,
    bp3)


def kernel(x, nlp_features, edge_index, user_indices,
           W_in, b_in, W_g1, b_g1, W_g2, b_g2,
           W_proj, b_proj, W_p1, b_p1, W_p2, b_p2, W_p3, b_p3):
  f32 = jnp.float32
  src = edge_index[0].astype(jnp.int32)
  dst = edge_index[1].astype(jnp.int32)
  pad = E_PAD - N_EDGES
  pad_iota = jnp.arange(pad, dtype=jnp.int32)
  src_p = jnp.concatenate(
      [src, pad_iota % N_NODES]).reshape(NW * EB, 128)
  dst_p = jnp.concatenate(
      [dst, TRASH_ROW + pad_iota % (AGG_ROWS - N_NODES)]).reshape(NW * EB, 128)
  sd = jnp.stack([src_p, dst_p])
  uidx = user_indices.astype(jnp.int32)

  h0 = _tc_linrelu(x, W_in, b_in.reshape(1, D))
  agg1 = _segsum_full(h0, sd)
  h1 = _tc_fuse(h0, agg1, W_g1, b_g1.reshape(1, D))
  uh1, uagg = _segsum_users(h1, sd, uidx)

  nlp_p = jnp.zeros((8, 896), f32).at[0, :NLP_DIM].set(nlp_features)
  wp1b = jnp.zeros((896, 256), f32).at[:NLP_DIM].set(W_p1[D:])
  wp3 = jnp.zeros((128, 128), f32).at[:, :1].set(W_p3)
  bp3 = jnp.zeros((1, 128), f32).at[0, 0].set(b_p3[0])

  out = _tc_head(uh1, uagg[0], uagg[1], W_g2, b_g2.reshape(1, D),
                 W_proj, b_proj.reshape(1, D), W_p1[:D], nlp_p, wp1b,
                 b_p1.reshape(1, 256), W_p2, b_p2.reshape(1, 128), wp3, bp3)
  return out[:, 0]


# speedup vs baseline: 10.3952x; 1.0090x over previous
import functools

import jax
import jax.numpy as jnp
from jax import lax
from jax.experimental import pallas as pl
from jax.experimental.pallas import tpu as pltpu
from jax.experimental.pallas import tpu_sc as plsc

N_NODES = 10000
D = 128
N_EDGES = 320000
N_USERS = 1024
NLP_DIM = 786

NC, NS = 2, 16
NW = NC * NS
EB = 80
E_PAD = NW * EB * 128
AGG_ROWS = 10112
RPT = AGG_ROWS // NS
TRASH_ROW = N_NODES
UPT = N_USERS // NS

_sc_mesh = plsc.VectorSubcoreMesh(core_axis_name="c", subcore_axis_name="s")


def _zero_accumulator(rows_v, agg_sh, s):
  def zrow(r, carry):
    for k in range(D // 16):
      rows_v[r, pl.ds(k * 16, 16)] = jnp.zeros((16,), jnp.float32)
    return carry
  lax.fori_loop(0, 128, zrow, 0)

  def zcopy(i, carry):
    pltpu.sync_copy(rows_v, agg_sh.at[pl.ds(s * RPT + i * 128, 128)])
    return carry
  lax.fori_loop(0, RPT // 128, zcopy, 0)
  rem = RPT % 128
  if rem:
    pltpu.sync_copy(rows_v.at[pl.ds(0, rem)],
                    agg_sh.at[pl.ds(s * RPT + RPT - rem, rem)])


NQ = 5
QB = EB // NQ


def _scatter_edges(h_hbm, sd_hbm, idx_a, idx_b, rows0, rows1,
                   agg_sh, sem0, sem1, sem_i, wid):
  bufs = (idx_a, idx_b)
  pltpu.sync_copy(sd_hbm.at[:, pl.ds(wid * EB, QB)], idx_a)
  pltpu.async_copy(h_hbm.at[idx_a.at[0, 0]], rows0, sem0)
  for q in range(NQ):
    cur = bufs[q % 2]
    nxt = bufs[(q + 1) % 2]
    if q + 1 < NQ:
      pltpu.async_copy(sd_hbm.at[:, pl.ds(wid * EB + (q + 1) * QB, QB)],
                       nxt, sem_i)

    def pair(p, carry, cur=cur):
      j0 = 2 * p
      j1 = 2 * p + 1
      j2 = 2 * p + 2
      pltpu.make_async_copy(h_hbm.at[cur.at[0, 0]], rows0, sem0).wait()
      pltpu.async_copy(h_hbm.at[cur.at[0, j1]], rows1, sem1)
      pltpu.sync_copy(rows0, agg_sh.at[cur.at[1, j0]], add=True)
      pltpu.make_async_copy(h_hbm.at[cur.at[0, 0]], rows1, sem1).wait()
      pltpu.async_copy(h_hbm.at[cur.at[0, j2]], rows0, sem0)
      pltpu.sync_copy(rows1, agg_sh.at[cur.at[1, j1]], add=True)
      return carry
    lax.fori_loop(0, QB // 2 - 1, pair, 0)
    pltpu.make_async_copy(h_hbm.at[cur.at[0, 0]], rows0, sem0).wait()
    pltpu.async_copy(h_hbm.at[cur.at[0, QB - 1]], rows1, sem1)
    pltpu.sync_copy(rows0, agg_sh.at[cur.at[1, QB - 2]], add=True)
    pltpu.make_async_copy(h_hbm.at[cur.at[0, 0]], rows1, sem1).wait()
    if q + 1 < NQ:
      pltpu.make_async_copy(sd_hbm.at[:, pl.ds(0, QB)], nxt, sem_i).wait()
      pltpu.async_copy(h_hbm.at[nxt.at[0, 0]], rows0, sem0)
    pltpu.sync_copy(rows1, agg_sh.at[cur.at[1, QB - 1]], add=True)


@functools.partial(
    pl.kernel,
    out_type=jax.ShapeDtypeStruct((NC, AGG_ROWS, D), jnp.float32),
    mesh=_sc_mesh,
    scratch_types=[
        pltpu.VMEM((2, QB, 128), jnp.int32),
        pltpu.VMEM((2, QB, 128), jnp.int32),
        pltpu.VMEM((128, D), jnp.float32),
        pltpu.VMEM((128, D), jnp.float32),
        pltpu.VMEM_SHARED((AGG_ROWS, D), jnp.float32),
        pltpu.SemaphoreType.DMA,
        pltpu.SemaphoreType.DMA,
        pltpu.SemaphoreType.DMA,
    ],
)
def _segsum_full(h_hbm, sd_hbm, out_hbm,
                 idx_a, idx_b, rows0, rows1, agg_sh, sem0, sem1, sem_i):
  c = lax.axis_index("c")
  s = lax.axis_index("s")
  wid = c * NS + s
  _zero_accumulator(rows0, agg_sh, s)
  plsc.subcore_barrier()
  _scatter_edges(h_hbm, sd_hbm, idx_a, idx_b, rows0, rows1,
                 agg_sh, sem0, sem1, sem_i, wid)
  plsc.subcore_barrier()
  pltpu.sync_copy(agg_sh.at[pl.ds(s * RPT, RPT)],
                  out_hbm.at[c, pl.ds(s * RPT, RPT)])


@functools.partial(
    pl.kernel,
    out_type=(jax.ShapeDtypeStruct((N_USERS, D), jnp.float32),
              jax.ShapeDtypeStruct((NC, N_USERS, D), jnp.float32)),
    mesh=_sc_mesh,
    scratch_types=[
        pltpu.VMEM((2, QB, 128), jnp.int32),
        pltpu.VMEM((2, QB, 128), jnp.int32),
        pltpu.VMEM((128, D), jnp.float32),
        pltpu.VMEM((128, D), jnp.float32),
        pltpu.VMEM((UPT,), jnp.int32),
        pltpu.VMEM_SHARED((AGG_ROWS, D), jnp.float32),
        pltpu.SemaphoreType.DMA,
        pltpu.SemaphoreType.DMA,
        pltpu.SemaphoreType.DMA,
    ],
)
def _segsum_users(h_hbm, sd_hbm, uidx_hbm, uh_hbm, uagg_hbm,
                  idx_a, idx_b, rows0, rows1, uidx_v, agg_sh,
                  sem0, sem1, sem_i):
  c = lax.axis_index("c")
  s = lax.axis_index("s")
  wid = c * NS + s
  _zero_accumulator(rows0, agg_sh, s)
  plsc.subcore_barrier()
  _scatter_edges(h_hbm, sd_hbm, idx_a, idx_b, rows0, rows1,
                 agg_sh, sem0, sem1, sem_i, wid)
  plsc.subcore_barrier()
  pltpu.sync_copy(uidx_hbm.at[pl.ds(s * UPT, UPT)], uidx_v)
  urows_v = rows0.at[pl.ds(0, UPT)]
  pltpu.async_copy(agg_sh.at[uidx_v], urows_v, sem0).wait()
  pltpu.sync_copy(urows_v, uagg_hbm.at[c, pl.ds(s * UPT, UPT)])

  @pl.when(c == 0)
  def _():
    pltpu.async_copy(h_hbm.at[uidx_v], urows_v, sem0).wait()
    pltpu.sync_copy(urows_v, uh_hbm.at[pl.ds(s * UPT, UPT)])


_NB = 2
_BR = N_NODES // _NB


def _linrelu_body(x_ref, w_ref, b_ref, o_ref):
  o_ref[...] = jnp.maximum(
      jnp.dot(x_ref[...], w_ref[...], preferred_element_type=jnp.float32)
      + b_ref[...], 0.0)


def _tc_linrelu(x, w, b):
  return pl.pallas_call(
      _linrelu_body,
      grid=(_NB,),
      in_specs=[
          pl.BlockSpec((_BR, D), lambda i: (i, 0)),
          pl.BlockSpec((D, D), lambda i: (0, 0)),
          pl.BlockSpec((1, D), lambda i: (0, 0)),
      ],
      out_specs=pl.BlockSpec((_BR, D), lambda i: (i, 0)),
      out_shape=jax.ShapeDtypeStruct((N_NODES, D), jnp.float32),
  )(x, w, b)


def _fuse_body(h_ref, a0_ref, a1_ref, w_ref, b_ref, o_ref):
  u = h_ref[...] + a0_ref[0] + a1_ref[0]
  o_ref[...] = jnp.maximum(
      jnp.dot(u, w_ref[...], preferred_element_type=jnp.float32)
      + b_ref[...], 0.0)


def _tc_fuse(h, agg, w, b):
  return pl.pallas_call(
      _fuse_body,
      grid=(_NB,),
      in_specs=[
          pl.BlockSpec((_BR, D), lambda i: (i, 0)),
          pl.BlockSpec((1, _BR, D), lambda i: (0, i, 0)),
          pl.BlockSpec((1, _BR, D), lambda i: (1, i, 0)),
          pl.BlockSpec((D, D), lambda i: (0, 0)),
          pl.BlockSpec((1, D), lambda i: (0, 0)),
      ],
      out_specs=pl.BlockSpec((_BR, D), lambda i: (i, 0)),
      out_shape=jax.ShapeDtypeStruct((N_NODES, D), jnp.float32),
  )(h, agg, agg, w, b)


def _head_body(uh_ref, a0_ref, a1_ref, wg2_ref, bg2_ref, wpj_ref, bpj_ref,
               wp1a_ref, nlp_ref, wp1b_ref, bp1_ref, wp2_ref, bp2_ref,
               wp3_ref, bp3_ref, o_ref):
  f32 = jnp.float32
  u = uh_ref[...] + a0_ref[...] + a1_ref[...]
  h2 = jnp.maximum(
      jnp.dot(u, wg2_ref[...], preferred_element_type=f32) + bg2_ref[...], 0.0)
  emb = jnp.dot(h2, wpj_ref[...], preferred_element_type=f32) + bpj_ref[...]
  nz = jnp.dot(nlp_ref[...], wp1b_ref[...], preferred_element_type=f32)[0:1, :]
  z1 = jnp.maximum(
      jnp.dot(emb, wp1a_ref[...], preferred_element_type=f32)
      + nz + bp1_ref[...], 0.0)
  z2 = jnp.maximum(
      jnp.dot(z1, wp2_ref[...], preferred_element_type=f32) + bp2_ref[...],
      0.0)
  lg = jnp.dot(z2, wp3_ref[...], preferred_element_type=f32) + bp3_ref[...]
  o_ref[...] = jax.nn.sigmoid(lg)


def _tc_head(uh, a0, a1, wg2, bg2, wpj, bpj, wp1a, nlp_p, wp1b, bp1, wp2,
             bp2, wp3, bp3):
  return pl.pallas_call(
      _head_body,
      out_shape=jax.ShapeDtypeStruct((N_USERS, 128), jnp.float32),
  )(uh, a0, a1, wg2, bg2, wpj, bpj, wp1a, nlp_p, wp1b, bp1, wp2, bp2, wp3,
    bp3)


def kernel(x, nlp_features, edge_index, user_indices,
           W_in, b_in, W_g1, b_g1, W_g2, b_g2,
           W_proj, b_proj, W_p1, b_p1, W_p2, b_p2, W_p3, b_p3):
  f32 = jnp.float32
  src = edge_index[0].astype(jnp.int32)
  dst = edge_index[1].astype(jnp.int32)
  pad = E_PAD - N_EDGES
  pad_iota = jnp.arange(pad, dtype=jnp.int32)
  src_p = jnp.concatenate(
      [src, pad_iota % N_NODES]).reshape(NW * EB, 128)
  dst_p = jnp.concatenate(
      [dst, TRASH_ROW + pad_iota % (AGG_ROWS - N_NODES)]).reshape(NW * EB, 128)
  sd = jnp.stack([src_p, dst_p])
  uidx = user_indices.astype(jnp.int32)

  h0 = _tc_linrelu(x, W_in, b_in.reshape(1, D))
  agg1 = _segsum_full(h0, sd)
  h1 = _tc_fuse(h0, agg1, W_g1, b_g1.reshape(1, D))
  uh1, uagg = _segsum_users(h1, sd, uidx)

  nlp_p = jnp.zeros((8, 896), f32).at[0, :NLP_DIM].set(nlp_features)
  wp1b = jnp.zeros((896, 256), f32).at[:NLP_DIM].set(W_p1[D:])
  wp3 = jnp.zeros((128, 128), f32).at[:, :1].set(W_p3)
  bp3 = jnp.zeros((1, 128), f32).at[0, 0].set(b_p3[0])

  out = _tc_head(uh1, uagg[0], uagg[1], W_g2, b_g2.reshape(1, D),
                 W_proj, b_proj.reshape(1, D), W_p1[:D], nlp_p, wp1b,
                 b_p1.reshape(1, 256), W_p2, b_p2.reshape(1, 128), wp3, bp3)
  return out[:, 0]


# speedup vs baseline: 13.0912x; 1.2594x over previous
import functools

import jax
import jax.numpy as jnp
from jax import lax
from jax.experimental import pallas as pl
from jax.experimental.pallas import tpu as pltpu
from jax.experimental.pallas import tpu_sc as plsc

N_NODES = 10000
D = 128
N_EDGES = 320000
N_USERS = 1024
NLP_DIM = 786

NC, NS = 2, 16
NW = NC * NS
EB = 80
E_PAD = NW * EB * 128
AGG_ROWS = 10112
RPT = AGG_ROWS // NS
TRASH_ROW = N_NODES
UPT = N_USERS // NS

_sc_mesh = plsc.VectorSubcoreMesh(core_axis_name="c", subcore_axis_name="s")


def _zero_accumulator(rows_v, agg_sh, s):
  def zrow(r, carry):
    for k in range(D // 16):
      rows_v[r, pl.ds(k * 16, 16)] = jnp.zeros((16,), jnp.float32)
    return carry
  lax.fori_loop(0, 64, zrow, 0)

  def zcopy(i, carry):
    pltpu.sync_copy(rows_v, agg_sh.at[pl.ds(s * RPT + i * 64, 64)])
    return carry
  lax.fori_loop(0, RPT // 64, zcopy, 0)
  rem = RPT % 64
  if rem:
    pltpu.sync_copy(rows_v.at[pl.ds(0, rem)],
                    agg_sh.at[pl.ds(s * RPT + RPT - rem, rem)])


NQ = 5
CH = 64
CPB = 32
CPT = NQ * CPB
NDEEP = 4


def _scatter_edges(h_hbm, sd_hbm, idx_a, idx_b, rowbufs,
                   agg_sh, sems, sem_i, wid):
  ibufs = (idx_a, idx_b)
  base = wid * CPT
  pltpu.sync_copy(sd_hbm.at[:, pl.ds(base, CPB)], idx_a)
  for k in range(NDEEP - 1):
    pltpu.async_copy(h_hbm.at[idx_a.at[0, k]], rowbufs[k], sems[k])
  for q in range(NQ):
    cur = ibufs[q % 2]
    nxt = ibufs[(q + 1) % 2]
    if q + 1 < NQ:
      pltpu.async_copy(sd_hbm.at[:, pl.ds(base + (q + 1) * CPB, CPB)],
                       nxt, sem_i)

    def quad(p, carry, cur=cur):
      for k in range(NDEEP):
        jk = NDEEP * p + k
        bw = rowbufs[k]
        bn = rowbufs[(k + NDEEP - 1) % NDEEP]
        pltpu.make_async_copy(h_hbm.at[cur.at[0, 0]], bw, sems[k]).wait()
        pltpu.async_copy(h_hbm.at[cur.at[0, jk + NDEEP - 1]], bn,
                         sems[(k + NDEEP - 1) % NDEEP])
        pltpu.sync_copy(bw, agg_sh.at[cur.at[1, jk]], add=True)
      return carry
    lax.fori_loop(0, CPB // NDEEP - 1, quad, 0)
    if q + 1 < NQ:
      pltpu.make_async_copy(sd_hbm.at[:, pl.ds(0, CPB)], nxt, sem_i).wait()
    for k in range(NDEEP):
      jk = CPB - NDEEP + k
      bw = rowbufs[k]
      bn = rowbufs[(k + NDEEP - 1) % NDEEP]
      sn = sems[(k + NDEEP - 1) % NDEEP]
      pltpu.make_async_copy(h_hbm.at[cur.at[0, 0]], bw, sems[k]).wait()
      if k == 0:
        pltpu.async_copy(h_hbm.at[cur.at[0, CPB - 1]], bn, sn)
      elif q + 1 < NQ:
        pltpu.async_copy(h_hbm.at[nxt.at[0, k - 1]], bn, sn)
      pltpu.sync_copy(bw, agg_sh.at[cur.at[1, jk]], add=True)


@functools.partial(
    pl.kernel,
    out_type=jax.ShapeDtypeStruct((NC, AGG_ROWS, D), jnp.float32),
    mesh=_sc_mesh,
    scratch_types=[
        pltpu.VMEM((2, CPB, CH), jnp.int32),
        pltpu.VMEM((2, CPB, CH), jnp.int32),
        pltpu.VMEM((CH, D), jnp.float32),
        pltpu.VMEM((CH, D), jnp.float32),
        pltpu.VMEM((CH, D), jnp.float32),
        pltpu.VMEM((CH, D), jnp.float32),
        pltpu.VMEM_SHARED((AGG_ROWS, D), jnp.float32),
        pltpu.SemaphoreType.DMA,
        pltpu.SemaphoreType.DMA,
        pltpu.SemaphoreType.DMA,
        pltpu.SemaphoreType.DMA,
        pltpu.SemaphoreType.DMA,
    ],
)
def _segsum_full(h_hbm, sd_hbm, out_hbm,
                 idx_a, idx_b, r0, r1, r2, r3, agg_sh,
                 sem0, sem1, sem2, sem3, sem_i):
  c = lax.axis_index("c")
  s = lax.axis_index("s")
  wid = c * NS + s
  _zero_accumulator(r0, agg_sh, s)
  plsc.subcore_barrier()
  _scatter_edges(h_hbm, sd_hbm, idx_a, idx_b, (r0, r1, r2, r3),
                 agg_sh, (sem0, sem1, sem2, sem3), sem_i, wid)
  plsc.subcore_barrier()
  pltpu.sync_copy(agg_sh.at[pl.ds(s * RPT, RPT)],
                  out_hbm.at[c, pl.ds(s * RPT, RPT)])


@functools.partial(
    pl.kernel,
    out_type=(jax.ShapeDtypeStruct((N_USERS, D), jnp.float32),
              jax.ShapeDtypeStruct((NC, N_USERS, D), jnp.float32)),
    mesh=_sc_mesh,
    scratch_types=[
        pltpu.VMEM((2, CPB, CH), jnp.int32),
        pltpu.VMEM((2, CPB, CH), jnp.int32),
        pltpu.VMEM((CH, D), jnp.float32),
        pltpu.VMEM((CH, D), jnp.float32),
        pltpu.VMEM((CH, D), jnp.float32),
        pltpu.VMEM((CH, D), jnp.float32),
        pltpu.VMEM((UPT,), jnp.int32),
        pltpu.VMEM_SHARED((AGG_ROWS, D), jnp.float32),
        pltpu.SemaphoreType.DMA,
        pltpu.SemaphoreType.DMA,
        pltpu.SemaphoreType.DMA,
        pltpu.SemaphoreType.DMA,
        pltpu.SemaphoreType.DMA,
    ],
)
def _segsum_users(h_hbm, sd_hbm, uidx_hbm, uh_hbm, uagg_hbm,
                  idx_a, idx_b, r0, r1, r2, r3, uidx_v, agg_sh,
                  sem0, sem1, sem2, sem3, sem_i):
  c = lax.axis_index("c")
  s = lax.axis_index("s")
  wid = c * NS + s
  _zero_accumulator(r0, agg_sh, s)
  plsc.subcore_barrier()
  _scatter_edges(h_hbm, sd_hbm, idx_a, idx_b, (r0, r1, r2, r3),
                 agg_sh, (sem0, sem1, sem2, sem3), sem_i, wid)
  plsc.subcore_barrier()
  pltpu.sync_copy(uidx_hbm.at[pl.ds(s * UPT, UPT)], uidx_v)
  pltpu.async_copy(agg_sh.at[uidx_v], r0, sem0).wait()
  pltpu.sync_copy(r0, uagg_hbm.at[c, pl.ds(s * UPT, UPT)])

  @pl.when(c == 0)
  def _():
    pltpu.async_copy(h_hbm.at[uidx_v], r0, sem0).wait()
    pltpu.sync_copy(r0, uh_hbm.at[pl.ds(s * UPT, UPT)])


_NB = 2
_BR = N_NODES // _NB


def _linrelu_body(x_ref, w_ref, b_ref, o_ref):
  o_ref[...] = jnp.maximum(
      jnp.dot(x_ref[...], w_ref[...], preferred_element_type=jnp.float32)
      + b_ref[...], 0.0)


def _tc_linrelu(x, w, b):
  return pl.pallas_call(
      _linrelu_body,
      grid=(_NB,),
      in_specs=[
          pl.BlockSpec((_BR, D), lambda i: (i, 0)),
          pl.BlockSpec((D, D), lambda i: (0, 0)),
          pl.BlockSpec((1, D), lambda i: (0, 0)),
      ],
      out_specs=pl.BlockSpec((_BR, D), lambda i: (i, 0)),
      out_shape=jax.ShapeDtypeStruct((N_NODES, D), jnp.float32),
  )(x, w, b)


def _fuse_body(h_ref, a0_ref, a1_ref, w_ref, b_ref, o_ref):
  u = h_ref[...] + a0_ref[0] + a1_ref[0]
  o_ref[...] = jnp.maximum(
      jnp.dot(u, w_ref[...], preferred_element_type=jnp.float32)
      + b_ref[...], 0.0)


def _tc_fuse(h, agg, w, b):
  return pl.pallas_call(
      _fuse_body,
      grid=(_NB,),
      in_specs=[
          pl.BlockSpec((_BR, D), lambda i: (i, 0)),
          pl.BlockSpec((1, _BR, D), lambda i: (0, i, 0)),
          pl.BlockSpec((1, _BR, D), lambda i: (1, i, 0)),
          pl.BlockSpec((D, D), lambda i: (0, 0)),
          pl.BlockSpec((1, D), lambda i: (0, 0)),
      ],
      out_specs=pl.BlockSpec((_BR, D), lambda i: (i, 0)),
      out_shape=jax.ShapeDtypeStruct((N_NODES, D), jnp.float32),
  )(h, agg, agg, w, b)


def _head_body(uh_ref, a0_ref, a1_ref, wg2_ref, bg2_ref, wpj_ref, bpj_ref,
               wp1a_ref, nlp_ref, wp1b_ref, bp1_ref, wp2_ref, bp2_ref,
               wp3_ref, bp3_ref, o_ref):
  f32 = jnp.float32
  u = uh_ref[...] + a0_ref[...] + a1_ref[...]
  h2 = jnp.maximum(
      jnp.dot(u, wg2_ref[...], preferred_element_type=f32) + bg2_ref[...], 0.0)
  emb = jnp.dot(h2, wpj_ref[...], preferred_element_type=f32) + bpj_ref[...]
  nz = jnp.dot(nlp_ref[...], wp1b_ref[...], preferred_element_type=f32)[0:1, :]
  z1 = jnp.maximum(
      jnp.dot(emb, wp1a_ref[...], preferred_element_type=f32)
      + nz + bp1_ref[...], 0.0)
  z2 = jnp.maximum(
      jnp.dot(z1, wp2_ref[...], preferred_element_type=f32) + bp2_ref[...],
      0.0)
  lg = jnp.dot(z2, wp3_ref[...], preferred_element_type=f32) + bp3_ref[...]
  o_ref[...] = jax.nn.sigmoid(lg)


def _tc_head(uh, a0, a1, wg2, bg2, wpj, bpj, wp1a, nlp_p, wp1b, bp1, wp2,
             bp2, wp3, bp3):
  return pl.pallas_call(
      _head_body,
      out_shape=jax.ShapeDtypeStruct((N_USERS, 128), jnp.float32),
  )(uh, a0, a1, wg2, bg2, wpj, bpj, wp1a, nlp_p, wp1b, bp1, wp2, bp2, wp3,
    bp3)


def kernel(x, nlp_features, edge_index, user_indices,
           W_in, b_in, W_g1, b_g1, W_g2, b_g2,
           W_proj, b_proj, W_p1, b_p1, W_p2, b_p2, W_p3, b_p3):
  f32 = jnp.float32
  src = edge_index[0].astype(jnp.int32)
  dst = edge_index[1].astype(jnp.int32)
  pad = E_PAD - N_EDGES
  pad_iota = jnp.arange(pad, dtype=jnp.int32)
  src_p = jnp.concatenate(
      [src, pad_iota % N_NODES]).reshape(NW * CPT, CH)
  dst_p = jnp.concatenate(
      [dst, TRASH_ROW + pad_iota % (AGG_ROWS - N_NODES)]).reshape(NW * CPT, CH)
  sd = jnp.stack([src_p, dst_p])
  uidx = user_indices.astype(jnp.int32)

  h0 = _tc_linrelu(x, W_in, b_in.reshape(1, D))
  agg1 = _segsum_full(h0, sd)
  h1 = _tc_fuse(h0, agg1, W_g1, b_g1.reshape(1, D))
  uh1, uagg = _segsum_users(h1, sd, uidx)

  nlp_p = jnp.zeros((8, 896), f32).at[0, :NLP_DIM].set(nlp_features)
  wp1b = jnp.zeros((896, 256), f32).at[:NLP_DIM].set(W_p1[D:])
  wp3 = jnp.zeros((128, 128), f32).at[:, :1].set(W_p3)
  bp3 = jnp.zeros((1, 128), f32).at[0, 0].set(b_p3[0])

  out = _tc_head(uh1, uagg[0], uagg[1], W_g2, b_g2.reshape(1, D),
                 W_proj, b_proj.reshape(1, D), W_p1[:D], nlp_p, wp1b,
                 b_p1.reshape(1, 256), W_p2, b_p2.reshape(1, 128), wp3, bp3)
  return out[:, 0]
